# baseline (speedup 1.0000x reference)
import jax
import jax.numpy as jnp
from jax.experimental import pallas as pl


def _mlp(inp, W1, b1, W2, b2):
    h = jax.nn.relu(inp @ W1 + b1)
    return h @ W2 + b2


def _block(x, ea, src, dst, p, pre):
    N = x.shape[0]
    m_in = jnp.concatenate([x[dst], x[src], ea], axis=1)
    msg = _mlp(m_in, p[pre + 'mW1'], p[pre + 'mb1'], p[pre + 'mW2'], p[pre + 'mb2'])
    cnt = jax.ops.segment_sum(jnp.ones((msg.shape[0],), msg.dtype), dst, num_segments=N)[:, None]
    cntc = jnp.maximum(cnt, 1.0)
    ssum = jax.ops.segment_sum(msg, dst, num_segments=N)
    mean = ssum / cntc
    mx = jax.ops.segment_max(msg, dst, num_segments=N)
    mx = jnp.where(cnt > 0, mx, 0.0)
    mn = -jax.ops.segment_max(-msg, dst, num_segments=N)
    mn = jnp.where(cnt > 0, mn, 0.0)
    msq = jax.ops.segment_sum(msg * msg, dst, num_segments=N) / cntc
    std = jnp.sqrt(jax.nn.relu(msq - mean * mean) + 1e-5)
    aggr = jnp.concatenate([mean, mn, mx, std], axis=-1)
    upd = _mlp(jnp.concatenate([aggr, x], axis=1),
               p[pre + 'uW1'], p[pre + 'ub1'], p[pre + 'uW2'], p[pre + 'ub2'])
    return msg, upd


def _readout_body(y_ref, w1_ref, b1_ref, w2_ref, b2_ref, w3_ref, b3_ref, out_ref):
    y = y_ref[...]
    h = jnp.maximum(y @ w1_ref[...] + b1_ref[...], 0.0)
    h = jnp.maximum(h @ w2_ref[...] + b2_ref[...], 0.0)
    out_ref[...] = h @ w3_ref[...] + b3_ref[...]


def kernel(x, edge_index, edge_attr, global_features, vertex_batch_map, edge_batch_map, params):
    src = edge_index[0]
    dst = edge_index[1]
    h = x
    ea = edge_attr
    for l in range(5):
        ea, h = _block(h, ea, src, dst, params, f'b{l}_')
    vertex_embeddings = h
    g = global_features[vertex_batch_map]
    y = jnp.concatenate([vertex_embeddings, g], axis=1)

    N = y.shape[0]
    NP = ((N + 127) // 128) * 128
    y_p = jnp.zeros((NP, y.shape[1]), y.dtype).at[:N].set(y)
    fc3W = params['fc3W']
    fc3b = params['fc3b']
    out = pl.pallas_call(
        _readout_body,
        out_shape=jax.ShapeDtypeStruct((NP, 128), jnp.float32),
        grid=(NP // 1280 if NP % 1280 == 0 else 1,),
        in_specs=[pl.BlockSpec((NP, y.shape[1]), lambda i: (0, 0))] + [pl.BlockSpec(None)] * 6,
        out_specs=pl.BlockSpec((NP, 128), lambda i: (0, 0)),
    )(y_p, params['fc1W'], params['fc1b'], params['fc2W'], params['fc2b'],
      jnp.pad(fc3W, ((0, 0), (0, 127))), jnp.pad(fc3b, (0, 127)))
    q_values = out[:N, :1]
    return (vertex_embeddings, q_values)


# speedup vs baseline: 2.5140x; 2.5140x over previous
import functools

import jax
import jax.numpy as jnp
from jax import lax
from jax.experimental import pallas as pl
from jax.experimental.pallas import tpu as pltpu
from jax.experimental.pallas import tpu_sc as plsc

_N = 10000
_E = 320000
_H = 64
_NW = 32
_NPW = 313
_NP = _NW * _NPW
_EPW = _E // _NW
_EPAD = 11024
_CH = 128
_NEG = -3.4e38
_POS = 3.4e38


def _wid():
    return lax.axis_index("s") * 2 + lax.axis_index("c")


def _mesh():
    return plsc.VectorSubcoreMesh(core_axis_name="c", subcore_axis_name="s")


_SC_PARAMS = pltpu.CompilerParams(needs_layout_passes=False, use_tc_tiling_on_sc=False)


def _bucketize_body(dst_hbm, peid_hbm, pdst_hbm, offs_hbm, cnt16_hbm,
                    stage, hist, cur, offsv, leid, ldst):
    w = _wid()
    base = w * _EPW
    lanes = lax.iota(jnp.int32, 16)
    zeros16 = jnp.zeros((16,), jnp.int32)
    ones16 = jnp.ones((16,), jnp.int32)

    def _zero_hist(i, _):
        hist[pl.ds(i * 16, 16)] = zeros16
        return 0
    lax.fori_loop(0, 32, _zero_hist, 0)

    def _zero_lists(i, _):
        leid[pl.ds(i * 16, 16)] = zeros16
        ldst[pl.ds(i * 16, 16)] = zeros16
        return 0
    lax.fori_loop(0, _EPAD // 16, _zero_lists, 0)

    def _chunk1(c, _):
        pltpu.sync_copy(dst_hbm.at[pl.ds(base + c * 2000, 2000)], stage.at[pl.ds(0, 2000)])
        def _vreg(k, _):
            v = stage[pl.ds(k * 16, 16)]
            bkt = lax.div(v, _NPW)
            plsc.addupdate_scatter(hist, [bkt * 16 + lanes], ones16)
            return 0
        lax.fori_loop(0, 125, _vreg, 0)
        return 0
    lax.fori_loop(0, 5, _chunk1, 0)

    carry = jnp.int32(0)
    for b in range(32):
        hv = hist[pl.ds(b * 16, 16)]
        cs = plsc.cumsum(hv)
        ex = cs - hv
        cur[pl.ds(b * 16, 16)] = ex + carry
        plsc.store_scatter(offsv, [jnp.full((16,), b, jnp.int32)],
                           jnp.full((16,), 1, jnp.int32) * carry,
                           mask=lanes == 0)
        tot = cs[15]
        carry = lax.div(carry + tot + 15, 16) * 16

    def _chunk2(c, _):
        pltpu.sync_copy(dst_hbm.at[pl.ds(base + c * 2000, 2000)], stage.at[pl.ds(0, 2000)])
        def _vreg(k, _):
            v = stage[pl.ds(k * 16, 16)]
            bkt = lax.div(v, _NPW)
            key = bkt * 16 + lanes
            pos = plsc.load_gather(cur, [key])
            eid = jnp.full((16,), base + c * 2000 + k * 16, jnp.int32) + lanes
            plsc.store_scatter(leid, [pos], eid)
            plsc.store_scatter(ldst, [pos], v)
            plsc.store_scatter(cur, [key], pos + 1)
            return 0
        lax.fori_loop(0, 125, _vreg, 0)
        return 0
    lax.fori_loop(0, 5, _chunk2, 0)

    pltpu.sync_copy(leid, peid_hbm.at[w])
    pltpu.sync_copy(ldst, pdst_hbm.at[w])
    pltpu.sync_copy(offsv, offs_hbm.at[w])
    pltpu.sync_copy(hist, cnt16_hbm.at[w])


def _bucketize(dst):
    return pl.kernel(
        _bucketize_body,
        out_type=(
            jax.ShapeDtypeStruct((_NW, _EPAD), jnp.int32),
            jax.ShapeDtypeStruct((_NW, _EPAD), jnp.int32),
            jax.ShapeDtypeStruct((_NW, 32), jnp.int32),
            jax.ShapeDtypeStruct((_NW, 512), jnp.int32),
        ),
        mesh=_mesh(),
        compiler_params=_SC_PARAMS,
        scratch_types=[
            pltpu.VMEM((2000,), jnp.int32),
            pltpu.VMEM((512,), jnp.int32),
            pltpu.VMEM((512,), jnp.int32),
            pltpu.VMEM((32,), jnp.int32),
            pltpu.VMEM((_EPAD,), jnp.int32),
            pltpu.VMEM((_EPAD,), jnp.int32),
        ],
    )(dst)


def _edge_gather_body(pd_hbm, ps_hbm, dst_hbm, src_hbm, g_hbm,
                      dstc, srcc, bufa, bufb, sema, semb):
    w = _wid()
    nch = jnp.where(w < 4, 79, 78)

    def _chunk(i, _):
        c = w + i * 32
        base = c * _CH
        pltpu.sync_copy(dst_hbm.at[pl.ds(base, _CH)], dstc)
        pltpu.sync_copy(src_hbm.at[pl.ds(base, _CH)], srcc)
        da = pltpu.async_copy(pd_hbm.at[dstc], bufa, sema)
        db = pltpu.async_copy(ps_hbm.at[srcc], bufb, semb)
        da.wait()
        db.wait()
        def _row(r, _):
            for j in range(4):
                bufa[r, pl.ds(j * 16, 16)] = (bufa[r, pl.ds(j * 16, 16)]
                                              + bufb[r, pl.ds(j * 16, 16)])
            return 0
        lax.fori_loop(0, _CH, _row, 0)
        pltpu.sync_copy(bufa, g_hbm.at[pl.ds(base, _CH)])
        return 0
    lax.fori_loop(0, nch, _chunk, 0)


def _edge_gather(pd, ps, dst, src):
    return pl.kernel(
        _edge_gather_body,
        out_type=jax.ShapeDtypeStruct((_E, _H), jnp.float32),
        mesh=_mesh(),
        compiler_params=_SC_PARAMS,
        scratch_types=[
            pltpu.VMEM((_CH,), jnp.int32),
            pltpu.VMEM((_CH,), jnp.int32),
            pltpu.VMEM((_CH, _H), jnp.float32),
            pltpu.VMEM((_CH, _H), jnp.float32),
            pltpu.SemaphoreType.DMA,
            pltpu.SemaphoreType.DMA,
        ],
    )(pd, ps, dst, src)


def _aggregate_body(msg_hbm, peid_hbm, pdst_hbm, offs_hbm, cnt16_hbm,
                    ssum_hbm, ssq_hbm, smx_hbm, smn_hbm, cntw_hbm,
                    accs, accq, accx, accn, cnt16v, cntf, eidc, dstc, gbuf,
                    offsv, c16, sem):
    b = _wid()
    nbase = b * _NPW
    lanes = lax.iota(jnp.int32, 16)
    zf = jnp.zeros((16,), jnp.float32)
    negv = jnp.full((16,), _NEG, jnp.float32)
    posv = jnp.full((16,), _POS, jnp.float32)

    def _init(i, _):
        accs[i, pl.ds(0, 16)] = zf
        accs[i, pl.ds(16, 16)] = zf
        accs[i, pl.ds(32, 16)] = zf
        accs[i, pl.ds(48, 16)] = zf
        accq[i, pl.ds(0, 16)] = zf
        accq[i, pl.ds(16, 16)] = zf
        accq[i, pl.ds(32, 16)] = zf
        accq[i, pl.ds(48, 16)] = zf
        accx[i, pl.ds(0, 16)] = negv
        accx[i, pl.ds(16, 16)] = negv
        accx[i, pl.ds(32, 16)] = negv
        accx[i, pl.ds(48, 16)] = negv
        accn[i, pl.ds(0, 16)] = posv
        accn[i, pl.ds(16, 16)] = posv
        accn[i, pl.ds(32, 16)] = posv
        accn[i, pl.ds(48, 16)] = posv
        return 0
    lax.fori_loop(0, _NPW, _init, 0)

    def _zc(i, _):
        cnt16v[pl.ds(i * 16, 16)] = jnp.zeros((16,), jnp.int32)
        return 0
    lax.fori_loop(0, _NPW, _zc, 0)

    for w in range(_NW):
        pltpu.sync_copy(offs_hbm.at[w], offsv)
        start = pl.multiple_of(plsc.load_gather(offsv, [jnp.full((16,), b, jnp.int32)])[0], 16)
        pltpu.sync_copy(cnt16_hbm.at[w].at[pl.ds(pl.multiple_of(b * 16, 16), 16)], c16)
        cnt_wb = jnp.sum(c16[...])
        nch = lax.div(cnt_wb + _CH - 1, _CH)

        def _chunk(c, _):
            off = pl.multiple_of(start + c * _CH, 16)
            pltpu.sync_copy(peid_hbm.at[w].at[pl.ds(off, _CH)], eidc)
            pltpu.sync_copy(pdst_hbm.at[w].at[pl.ds(off, _CH)],
                            dstc.at[pl.ds(0, _CH)])
            pltpu.async_copy(msg_hbm.at[eidc], gbuf, sem).wait()

            def _edge(e, _):
                t = plsc.load_gather(dstc, [jnp.full((16,), 0, jnp.int32) + e])[0] - nbase
                ok = (c * _CH + e) < cnt_wb
                t = jnp.where(ok, t, 0)
                for j in range(4):
                    row = gbuf[e, pl.ds(j * 16, 16)]
                    rz = jnp.where(ok, row, 0.0)
                    rx = jnp.where(ok, row, _NEG)
                    rn = jnp.where(ok, row, _POS)
                    s = accs[t, pl.ds(j * 16, 16)]
                    accs[t, pl.ds(j * 16, 16)] = s + rz
                    q = accq[t, pl.ds(j * 16, 16)]
                    accq[t, pl.ds(j * 16, 16)] = q + rz * rz
                    x = accx[t, pl.ds(j * 16, 16)]
                    accx[t, pl.ds(j * 16, 16)] = jnp.maximum(x, rx)
                    n = accn[t, pl.ds(j * 16, 16)]
                    accn[t, pl.ds(j * 16, 16)] = jnp.minimum(n, rn)
                return 0
            lax.fori_loop(0, _CH, _edge, 0)

            def _cvec(k, _):
                tv = dstc[pl.ds(k * 16, 16)] - nbase
                inr = (c * _CH + k * 16 + lanes) < cnt_wb
                tv = jnp.where(inr, tv, 0)
                plsc.addupdate_scatter(cnt16v, [tv * 16 + lanes],
                                       jnp.ones((16,), jnp.int32), mask=inr)
                return 0
            lax.fori_loop(0, _CH // 16, _cvec, 0)
            return 0
        lax.fori_loop(0, nch, _chunk, 0)

    def _csum(g, _):
        node = g * 16 + lanes
        tot = jnp.zeros((16,), jnp.int32)
        for l in range(16):
            tot = tot + plsc.load_gather(cnt16v, [node * 16 + l])
        cntf[pl.ds(g * 16, 16)] = tot.astype(jnp.float32)
        return 0
    lax.fori_loop(0, 20, _csum, 0)

    pltpu.sync_copy(accs, ssum_hbm.at[pl.ds(nbase, _NPW)])
    pltpu.sync_copy(accq, ssq_hbm.at[pl.ds(nbase, _NPW)])
    pltpu.sync_copy(accx, smx_hbm.at[pl.ds(nbase, _NPW)])
    pltpu.sync_copy(accn, smn_hbm.at[pl.ds(nbase, _NPW)])
    pltpu.sync_copy(cntf, cntw_hbm.at[b])


def _aggregate(msg, peid, pdst, offs, cnt16):
    return pl.kernel(
        _aggregate_body,
        out_type=(
            jax.ShapeDtypeStruct((_NP, _H), jnp.float32),
            jax.ShapeDtypeStruct((_NP, _H), jnp.float32),
            jax.ShapeDtypeStruct((_NP, _H), jnp.float32),
            jax.ShapeDtypeStruct((_NP, _H), jnp.float32),
            jax.ShapeDtypeStruct((_NW, 320), jnp.float32),
        ),
        mesh=_mesh(),
        compiler_params=_SC_PARAMS,
        scratch_types=[
            pltpu.VMEM((_NPW, _H), jnp.float32),
            pltpu.VMEM((_NPW, _H), jnp.float32),
            pltpu.VMEM((_NPW, _H), jnp.float32),
            pltpu.VMEM((_NPW, _H), jnp.float32),
            pltpu.VMEM((_NPW * 16,), jnp.int32),
            pltpu.VMEM((320,), jnp.float32),
            pltpu.VMEM((_CH,), jnp.int32),
            pltpu.VMEM((_CH + 16,), jnp.int32),
            pltpu.VMEM((_CH, _H), jnp.float32),
            pltpu.VMEM((32,), jnp.int32),
            pltpu.VMEM((16,), jnp.int32),
            pltpu.SemaphoreType.DMA,
        ],
    )(msg, peid, pdst, offs, cnt16)


def _proj_body(h_ref, wd_ref, ws_ref, b1_ref, pd_ref, ps_ref):
    h = h_ref[...]
    pd_ref[...] = jnp.dot(h, wd_ref[...], preferred_element_type=jnp.float32) + b1_ref[...]
    ps_ref[...] = jnp.dot(h, ws_ref[...], preferred_element_type=jnp.float32)


def _proj(h, wd, ws, b1):
    return pl.pallas_call(
        _proj_body,
        out_shape=(jax.ShapeDtypeStruct((_NP, _H), jnp.float32),
                   jax.ShapeDtypeStruct((_NP, _H), jnp.float32)),
    )(h, wd, ws, b1.reshape(1, _H))


def _edge_mlp_body(g_ref, ea_ref, we_ref, w2_ref, b2_ref, out_ref):
    pre = g_ref[...] + jnp.dot(ea_ref[...], we_ref[...], preferred_element_type=jnp.float32)
    h = jnp.maximum(pre, 0.0)
    out_ref[...] = jnp.dot(h, w2_ref[...], preferred_element_type=jnp.float32) + b2_ref[...]


def _edge_mlp(g, ea, we, w2, b2):
    be = 1280
    grid = _E // be
    return pl.pallas_call(
        _edge_mlp_body,
        out_shape=jax.ShapeDtypeStruct((_E, _H), jnp.float32),
        grid=(grid,),
        in_specs=[
            pl.BlockSpec((be, _H), lambda i: (i, 0)),
            pl.BlockSpec((be, _H), lambda i: (i, 0)),
            pl.BlockSpec((_H, _H), lambda i: (0, 0)),
            pl.BlockSpec((_H, _H), lambda i: (0, 0)),
            pl.BlockSpec((1, _H), lambda i: (0, 0)),
        ],
        out_specs=pl.BlockSpec((be, _H), lambda i: (i, 0)),
    )(g, ea, we, w2, b2.reshape(1, _H))


def _update_body(ssum_ref, ssq_ref, smx_ref, smn_ref, cnt_ref, h_ref,
                 um_ref, un_ref, ux_ref, us_ref, uh_ref, ub1_ref,
                 w2_ref, ub2_ref, out_ref):
    cnt = cnt_ref[...]
    cntc = jnp.maximum(cnt, 1.0)
    mean = ssum_ref[...] / cntc
    msq = ssq_ref[...] / cntc
    std = jnp.sqrt(jnp.maximum(msq - mean * mean, 0.0) + 1e-5)
    pos = cnt > 0.0
    mx = jnp.where(pos, smx_ref[...], 0.0)
    mn = jnp.where(pos, smn_ref[...], 0.0)
    z = (jnp.dot(mean, um_ref[...], preferred_element_type=jnp.float32)
         + jnp.dot(mn, un_ref[...], preferred_element_type=jnp.float32)
         + jnp.dot(mx, ux_ref[...], preferred_element_type=jnp.float32)
         + jnp.dot(std, us_ref[...], preferred_element_type=jnp.float32)
         + jnp.dot(h_ref[...], uh_ref[...], preferred_element_type=jnp.float32)
         + ub1_ref[...])
    z = jnp.maximum(z, 0.0)
    out_ref[...] = jnp.dot(z, w2_ref[...], preferred_element_type=jnp.float32) + ub2_ref[...]


def _update(ssum, ssq, smx, smn, cnt2d, h, um, un, ux, us, uh, ub1, w2, ub2):
    return pl.pallas_call(
        _update_body,
        out_shape=jax.ShapeDtypeStruct((_NP, _H), jnp.float32),
    )(ssum, ssq, smx, smn, cnt2d, h, um, un, ux, us, uh,
      ub1.reshape(1, _H), w2, ub2.reshape(1, _H))


def _readout_body(h_ref, vb_ref, gf_ref, f1h_ref, f1g_ref, b1_ref,
                  w2_ref, b2_ref, w3_ref, b3_ref, out_ref):
    onehot = (vb_ref[...] == lax.broadcasted_iota(jnp.int32, (1, 16), 1).astype(jnp.float32)).astype(jnp.float32)
    g = jnp.dot(onehot, gf_ref[...], preferred_element_type=jnp.float32)
    y = (jnp.dot(h_ref[...], f1h_ref[...], preferred_element_type=jnp.float32)
         + jnp.dot(g, f1g_ref[...], preferred_element_type=jnp.float32)
         + b1_ref[...])
    y = jnp.maximum(y, 0.0)
    y = jnp.maximum(jnp.dot(y, w2_ref[...], preferred_element_type=jnp.float32) + b2_ref[...], 0.0)
    out_ref[...] = jnp.dot(y, w3_ref[...], preferred_element_type=jnp.float32) + b3_ref[...]


def _readout(h, vb16, gf, f1h, f1g, b1, w2, b2, w3p, b3p):
    return pl.pallas_call(
        _readout_body,
        out_shape=jax.ShapeDtypeStruct((_NP, 128), jnp.float32),
    )(h, vb16, gf, f1h, f1g, b1.reshape(1, _H), w2, b2.reshape(1, _H),
      w3p, b3p.reshape(1, 128))


def kernel(x, edge_index, edge_attr, global_features, vertex_batch_map,
           edge_batch_map, params):
    src = edge_index[0].astype(jnp.int32)
    dst = edge_index[1].astype(jnp.int32)

    peid, pdst, offs, cnt16 = _bucketize(dst)

    h = jnp.pad(x, ((0, _NP - _N), (0, 0)))
    ea = jnp.pad(edge_attr, ((0, 0), (0, _H - edge_attr.shape[1])))

    for l in range(5):
        p = params
        pre = f'b{l}_'
        mW1 = p[pre + 'mW1']
        fi = mW1.shape[0] - _H if l > 0 else 2 * 128 + 16
        fdim = 128 if l == 0 else _H
        wd = mW1[:fdim]
        ws = mW1[fdim:2 * fdim]
        we = mW1[2 * fdim:]
        if l == 0:
            we = jnp.pad(we, ((0, _H - we.shape[0]), (0, 0)))
        pd, ps = _proj(h, wd, ws, p[pre + 'mb1'])
        g = _edge_gather(pd, ps, dst, src)
        msg = _edge_mlp(g, ea, we, p[pre + 'mW2'], p[pre + 'mb2'])
        ssum, ssq, smx, smn, cntw = _aggregate(msg, peid, pdst, offs, cnt16)
        cnt_full = cntw[:, :_NPW].reshape(_NP)
        cnt2d = jnp.broadcast_to(cnt_full[:, None], (_NP, _H))
        uW1 = p[pre + 'uW1']
        um = uW1[0:_H]
        un = uW1[_H:2 * _H]
        ux = uW1[2 * _H:3 * _H]
        us = uW1[3 * _H:4 * _H]
        uh = uW1[4 * _H:]
        h = _update(ssum, ssq, smx, smn, cnt2d, h, um, un, ux, us, uh,
                    p[pre + 'ub1'], p[pre + 'uW2'], p[pre + 'ub2'])
        ea = msg

    vertex_embeddings = h[:_N]

    vbp = jnp.pad(vertex_batch_map.astype(jnp.float32), (0, _NP - _N))
    vb16 = jnp.broadcast_to(vbp[:, None], (_NP, 16))
    w3p = jnp.pad(params['fc3W'], ((0, 0), (0, 127)))
    b3p = jnp.pad(params['fc3b'], (0, 127))
    q = _readout(h, vb16, global_features, params['fc1W'][:_H],
                 params['fc1W'][_H:], params['fc1b'], params['fc2W'],
                 params['fc2b'], w3p, b3p)
    q_values = q[:_N, :1]
    return (vertex_embeddings, q_values)


# speedup vs baseline: 3.4577x; 1.3754x over previous
import jax
import jax.numpy as jnp
from jax import lax
from jax.experimental import pallas as pl
from jax.experimental.pallas import tpu as pltpu
from jax.experimental.pallas import tpu_sc as plsc

_N = 10000
_E = 320000
_H = 64
_NW = 32
_NPW = 313
_NP = _NW * _NPW
_EPW = _E // _NW
_EPAD = 11024
_BLK = 384
_EPW2 = 28 * _BLK
_EP = _NW * _EPW2
_CAP2 = 12288
_CH = 128
_NEG = -3.4e38
_POS = 3.4e38


def _wid():
    return lax.axis_index("s") * 2 + lax.axis_index("c")


def _mesh():
    return plsc.VectorSubcoreMesh(core_axis_name="c", subcore_axis_name="s")


_SC_PARAMS = pltpu.CompilerParams(needs_layout_passes=False, use_tc_tiling_on_sc=False)


def _bucketize_body(dst_hbm, peid_hbm, pdst_hbm, offs_hbm, cnt16_hbm,
                    stage, hist, cur, offsv, leid, ldst):
    w = _wid()
    base = w * _EPW
    lanes = lax.iota(jnp.int32, 16)
    zeros16 = jnp.zeros((16,), jnp.int32)
    ones16 = jnp.ones((16,), jnp.int32)

    def _zero_hist(i, _):
        hist[pl.ds(i * 16, 16)] = zeros16
        return 0
    lax.fori_loop(0, 32, _zero_hist, 0)

    def _zero_lists(i, _):
        leid[pl.ds(i * 16, 16)] = zeros16
        ldst[pl.ds(i * 16, 16)] = zeros16
        return 0
    lax.fori_loop(0, _EPAD // 16, _zero_lists, 0)

    def _chunk1(c, _):
        pltpu.sync_copy(dst_hbm.at[pl.ds(base + c * 2000, 2000)], stage)
        def _vreg(k, _):
            v = stage[pl.ds(k * 16, 16)]
            bkt = lax.div(v, _NPW)
            plsc.addupdate_scatter(hist, [bkt * 16 + lanes], ones16)
            return 0
        lax.fori_loop(0, 125, _vreg, 0)
        return 0
    lax.fori_loop(0, 5, _chunk1, 0)

    carry = jnp.int32(0)
    for b in range(32):
        hv = hist[pl.ds(b * 16, 16)]
        cs = plsc.cumsum(hv)
        cur[pl.ds(b * 16, 16)] = cs - hv + carry
        plsc.store_scatter(offsv, [jnp.full((16,), b, jnp.int32)],
                           jnp.full((16,), 1, jnp.int32) * carry,
                           mask=lanes == 0)
        carry = lax.div(carry + cs[15] + 15, 16) * 16

    def _chunk2(c, _):
        pltpu.sync_copy(dst_hbm.at[pl.ds(base + c * 2000, 2000)], stage)
        def _vreg(k, _):
            v = stage[pl.ds(k * 16, 16)]
            bkt = lax.div(v, _NPW)
            key = bkt * 16 + lanes
            pos = plsc.load_gather(cur, [key])
            eid = jnp.full((16,), base + c * 2000 + k * 16, jnp.int32) + lanes
            plsc.store_scatter(leid, [pos], eid)
            plsc.store_scatter(ldst, [pos], v)
            plsc.store_scatter(cur, [key], pos + 1)
            return 0
        lax.fori_loop(0, 125, _vreg, 0)
        return 0
    lax.fori_loop(0, 5, _chunk2, 0)

    pltpu.sync_copy(leid, peid_hbm.at[w])
    pltpu.sync_copy(ldst, pdst_hbm.at[w])
    pltpu.sync_copy(offsv, offs_hbm.at[w])
    pltpu.sync_copy(hist, cnt16_hbm.at[w])


def _bucketize(dst):
    return pl.kernel(
        _bucketize_body,
        out_type=(
            jax.ShapeDtypeStruct((_NW, _EPAD), jnp.int32),
            jax.ShapeDtypeStruct((_NW, _EPAD), jnp.int32),
            jax.ShapeDtypeStruct((_NW, 32), jnp.int32),
            jax.ShapeDtypeStruct((_NW, 512), jnp.int32),
        ),
        mesh=_mesh(),
        compiler_params=_SC_PARAMS,
        scratch_types=[
            pltpu.VMEM((2000,), jnp.int32),
            pltpu.VMEM((512,), jnp.int32),
            pltpu.VMEM((512,), jnp.int32),
            pltpu.VMEM((32,), jnp.int32),
            pltpu.VMEM((_EPAD,), jnp.int32),
            pltpu.VMEM((_EPAD,), jnp.int32),
        ],
    )(dst)


def _sortlocal_body(peid_hbm, pdst_hbm, offs_hbm, cnt16_hbm,
                    seid_hbm, meta_hbm, tot16_hbm, cntw_hbm,
                    eidc, dstc, hist, cur, cntf, seidl, sdstl, offsv, c16, t16):
    b = _wid()
    nbase = b * _NPW
    lanes = lax.iota(jnp.int32, 16)
    zeros16 = jnp.zeros((16,), jnp.int32)
    ones16 = jnp.ones((16,), jnp.int32)

    def _zh(i, _):
        hist[pl.ds(i * 16, 16)] = zeros16
        return 0
    lax.fori_loop(0, _NPW, _zh, 0)

    def _zs(i, _):
        seidl[pl.ds(i * 16, 16)] = zeros16
        sdstl[pl.ds(i * 16, 16)] = zeros16
        return 0
    lax.fori_loop(0, (_CAP2 + 16) // 16, _zs, 0)

    for w in range(_NW):
        pltpu.sync_copy(offs_hbm.at[w], offsv)
        start = pl.multiple_of(
            plsc.load_gather(offsv, [jnp.full((16,), b, jnp.int32)])[0], 16)
        pltpu.sync_copy(cnt16_hbm.at[w].at[pl.ds(pl.multiple_of(b * 16, 16), 16)], c16)
        cnt_wb = jnp.sum(c16[...])
        nch = lax.div(cnt_wb + 511, 512)

        def _chunk(c, _):
            off = pl.multiple_of(start + c * 512, 16)
            pltpu.sync_copy(pdst_hbm.at[w].at[pl.ds(off, 512)], dstc)
            def _vreg(k, _):
                dv = dstc[pl.ds(k * 16, 16)]
                tv = jnp.clip(dv - nbase, 0, _NPW - 1)
                inr = (c * 512 + k * 16 + lanes) < cnt_wb
                plsc.addupdate_scatter(hist, [tv * 16 + lanes], ones16, mask=inr)
                return 0
            lax.fori_loop(0, 32, _vreg, 0)
            return 0
        lax.fori_loop(0, nch, _chunk, 0)

    def _scan(i, carry):
        hv = hist[pl.ds(i * 16, 16)]
        cs = plsc.cumsum(hv)
        cur[pl.ds(i * 16, 16)] = cs - hv + carry
        plsc.store_scatter(cntf, [jnp.full((16,), 0, jnp.int32) + i],
                           jnp.zeros((16,), jnp.float32) + cs[15].astype(jnp.float32),
                           mask=lanes == 0)
        return carry + cs[15]
    total = lax.fori_loop(0, _NPW, _scan, jnp.int32(0))

    for w in range(_NW):
        pltpu.sync_copy(offs_hbm.at[w], offsv)
        start = pl.multiple_of(
            plsc.load_gather(offsv, [jnp.full((16,), b, jnp.int32)])[0], 16)
        pltpu.sync_copy(cnt16_hbm.at[w].at[pl.ds(pl.multiple_of(b * 16, 16), 16)], c16)
        cnt_wb = jnp.sum(c16[...])
        nch = lax.div(cnt_wb + 511, 512)

        def _chunk(c, _):
            off = pl.multiple_of(start + c * 512, 16)
            pltpu.sync_copy(peid_hbm.at[w].at[pl.ds(off, 512)], eidc)
            pltpu.sync_copy(pdst_hbm.at[w].at[pl.ds(off, 512)], dstc)
            def _vreg(k, _):
                dv = dstc[pl.ds(k * 16, 16)]
                ev = eidc[pl.ds(k * 16, 16)]
                tv = jnp.clip(dv - nbase, 0, _NPW - 1)
                key = tv * 16 + lanes
                inr = (c * 512 + k * 16 + lanes) < cnt_wb
                pos = plsc.load_gather(cur, [key])
                plsc.store_scatter(seidl, [pos], ev, mask=inr)
                plsc.store_scatter(sdstl, [pos], dv, mask=inr)
                plsc.store_scatter(cur, [key], pos + 1, mask=inr)
                return 0
            lax.fori_loop(0, 32, _vreg, 0)
            return 0
        lax.fori_loop(0, nch, _chunk, 0)

    def _meta(i, _):
        dv = sdstl[pl.ds(i * 16, 16)]
        dn = plsc.load_gather(sdstl, [i * 16 + 1 + lanes])
        tv = jnp.clip(dv - nbase, 0, _NPW - 1)
        fl = jnp.where(dv != dn, jnp.int32(1 << 16), jnp.int32(0))
        sdstl[pl.ds(i * 16, 16)] = tv + fl
        return 0
    lax.fori_loop(0, _CAP2 // 16, _meta, 0)

    t16[...] = jnp.full((16,), 1, jnp.int32) * total
    pltpu.sync_copy(seidl, seid_hbm.at[b])
    pltpu.sync_copy(sdstl.at[pl.ds(0, _CAP2)], meta_hbm.at[b])
    pltpu.sync_copy(t16, tot16_hbm.at[b])
    pltpu.sync_copy(cntf, cntw_hbm.at[b])


def _sortlocal(peid, pdst, offs, cnt16):
    return pl.kernel(
        _sortlocal_body,
        out_type=(
            jax.ShapeDtypeStruct((_NW, _CAP2), jnp.int32),
            jax.ShapeDtypeStruct((_NW, _CAP2), jnp.int32),
            jax.ShapeDtypeStruct((_NW, 16), jnp.int32),
            jax.ShapeDtypeStruct((_NW, 320), jnp.float32),
        ),
        mesh=_mesh(),
        compiler_params=_SC_PARAMS,
        scratch_types=[
            pltpu.VMEM((512,), jnp.int32),
            pltpu.VMEM((512,), jnp.int32),
            pltpu.VMEM((_NPW * 16,), jnp.int32),
            pltpu.VMEM((_NPW * 16,), jnp.int32),
            pltpu.VMEM((320,), jnp.float32),
            pltpu.VMEM((_CAP2,), jnp.int32),
            pltpu.VMEM((_CAP2 + 16,), jnp.int32),
            pltpu.VMEM((32,), jnp.int32),
            pltpu.VMEM((16,), jnp.int32),
            pltpu.VMEM((16,), jnp.int32),
        ],
    )(peid, pdst, offs, cnt16)


def _edge_gather_body(pd_hbm, ps_hbm, dst_hbm, src_hbm, g_hbm,
                      dstb, srcb, bufa, bufb, sema, semb):
    w = _wid()
    ebase = w * _EPW2
    nblk = _EPW2 // _BLK

    pltpu.sync_copy(dst_hbm.at[pl.ds(ebase, _EPW2)], dstb)
    pltpu.sync_copy(src_hbm.at[pl.ds(ebase, _EPW2)], srcb)

    def _issue(t, q):
        for k in range(_BLK // _CH):
            off = pl.multiple_of(t * _BLK + k * _CH, _CH)
            sl = pl.ds(k * _CH, _CH)
            pltpu.async_copy(pd_hbm.at[dstb.at[pl.ds(off, _CH)]],
                             bufa.at[q].at[sl], sema.at[q])
            pltpu.async_copy(ps_hbm.at[srcb.at[pl.ds(off, _CH)]],
                             bufb.at[q].at[sl], semb.at[q])

    def _drain(q):
        for k in range(_BLK // _CH):
            sl = pl.ds(k * _CH, _CH)
            pltpu.make_async_copy(pd_hbm.at[dstb.at[pl.ds(0, _CH)]],
                                  bufa.at[q].at[sl], sema.at[q]).wait()
            pltpu.make_async_copy(ps_hbm.at[srcb.at[pl.ds(0, _CH)]],
                                  bufb.at[q].at[sl], semb.at[q]).wait()

    _issue(0, 0)

    def _step(t, _):
        q = lax.rem(t, 2)

        @pl.when(t + 1 < nblk)
        def _():
            _issue(t + 1, 1 - q)

        _drain(q)

        def _row(r, _):
            for j in range(4):
                sl = pl.ds(j * 16, 16)
                bufa[q, r, sl] = bufa[q, r, sl] + bufb[q, r, sl]
            return 0
        lax.fori_loop(0, _BLK, _row, 0)
        pltpu.sync_copy(bufa.at[q], g_hbm.at[pl.ds(ebase + t * _BLK, _BLK)])
        return 0
    lax.fori_loop(0, nblk, _step, 0)


def _edge_gather(pd, ps, dst, src):
    return pl.kernel(
        _edge_gather_body,
        out_type=jax.ShapeDtypeStruct((_EP, _H), jnp.float32),
        mesh=_mesh(),
        compiler_params=_SC_PARAMS,
        scratch_types=[
            pltpu.VMEM((_EPW2,), jnp.int32),
            pltpu.VMEM((_EPW2,), jnp.int32),
            pltpu.VMEM((2, _BLK, _H), jnp.float32),
            pltpu.VMEM((2, _BLK, _H), jnp.float32),
            pltpu.SemaphoreType.DMA((2,)),
            pltpu.SemaphoreType.DMA((2,)),
        ],
    )(pd, ps, dst, src)


def _aggregate_body(msg_hbm, seid_hbm, meta_hbm, tot16_hbm,
                    ssum_hbm, ssq_hbm, smx_hbm, smn_hbm,
                    accs, accq, accx, accn, seidl, metal, gbuf, t16, sem):
    b = _wid()
    nbase = b * _NPW
    zf = jnp.zeros((16,), jnp.float32)
    negv = jnp.full((16,), _NEG, jnp.float32)
    posv = jnp.full((16,), _POS, jnp.float32)

    def _init(i, _):
        for j in range(4):
            sl = pl.ds(j * 16, 16)
            accs[i, sl] = zf
            accq[i, sl] = zf
            accx[i, sl] = negv
            accn[i, sl] = posv
        return 0
    lax.fori_loop(0, _NPW, _init, 0)

    pltpu.sync_copy(seid_hbm.at[b], seidl)
    pltpu.sync_copy(meta_hbm.at[b], metal)
    pltpu.sync_copy(tot16_hbm.at[b], t16)
    total = t16[...][0]
    nblk = lax.div(total + _CH - 1, _CH)

    def _issue(t, q):
        off = pl.multiple_of(t * _CH, _CH)
        pltpu.async_copy(msg_hbm.at[seidl.at[pl.ds(off, _CH)]], gbuf.at[q], sem.at[q])

    def _drain(q):
        pltpu.make_async_copy(msg_hbm.at[seidl.at[pl.ds(0, _CH)]],
                              gbuf.at[q], sem.at[q]).wait()

    @pl.when(nblk > 0)
    def _():
        _issue(0, 0)

        def _blk(t, _):
            q = lax.rem(t, 2)

            @pl.when(t + 1 < nblk)
            def _():
                _issue(t + 1, 1 - q)

            _drain(q)

            def _edge(e, regs):
                (s0, s1, s2, s3, q0, q1, q2, q3,
                 x0, x1, x2, x3, n0, n1, n2, n3) = regs
                idx = t * _CH + e
                mv = plsc.load_gather(metal, [jnp.full((16,), 0, jnp.int32) + idx])[0]
                tnode = jnp.minimum(mv & 0xFFFF, _NPW - 1)
                ok = idx < total
                fl = (mv >= (1 << 16)) | (e == _CH - 1)
                r0 = gbuf[q, e, pl.ds(0, 16)]
                r1 = gbuf[q, e, pl.ds(16, 16)]
                r2 = gbuf[q, e, pl.ds(32, 16)]
                r3 = gbuf[q, e, pl.ds(48, 16)]
                z0 = jnp.where(ok, r0, 0.0)
                z1 = jnp.where(ok, r1, 0.0)
                z2 = jnp.where(ok, r2, 0.0)
                z3 = jnp.where(ok, r3, 0.0)
                s0 = s0 + z0
                s1 = s1 + z1
                s2 = s2 + z2
                s3 = s3 + z3
                q0 = q0 + z0 * z0
                q1 = q1 + z1 * z1
                q2 = q2 + z2 * z2
                q3 = q3 + z3 * z3
                x0 = jnp.maximum(x0, jnp.where(ok, r0, _NEG))
                x1 = jnp.maximum(x1, jnp.where(ok, r1, _NEG))
                x2 = jnp.maximum(x2, jnp.where(ok, r2, _NEG))
                x3 = jnp.maximum(x3, jnp.where(ok, r3, _NEG))
                n0 = jnp.minimum(n0, jnp.where(ok, r0, _POS))
                n1 = jnp.minimum(n1, jnp.where(ok, r1, _POS))
                n2 = jnp.minimum(n2, jnp.where(ok, r2, _POS))
                n3 = jnp.minimum(n3, jnp.where(ok, r3, _POS))

                @pl.when(fl)
                def _():
                    svs = (s0, s1, s2, s3)
                    qvs = (q0, q1, q2, q3)
                    xvs = (x0, x1, x2, x3)
                    nvs = (n0, n1, n2, n3)
                    for j in range(4):
                        sl = pl.ds(j * 16, 16)
                        accs[tnode, sl] = accs[tnode, sl] + svs[j]
                        accq[tnode, sl] = accq[tnode, sl] + qvs[j]
                        accx[tnode, sl] = jnp.maximum(accx[tnode, sl], xvs[j])
                        accn[tnode, sl] = jnp.minimum(accn[tnode, sl], nvs[j])

                s0 = jnp.where(fl, 0.0, s0)
                s1 = jnp.where(fl, 0.0, s1)
                s2 = jnp.where(fl, 0.0, s2)
                s3 = jnp.where(fl, 0.0, s3)
                q0 = jnp.where(fl, 0.0, q0)
                q1 = jnp.where(fl, 0.0, q1)
                q2 = jnp.where(fl, 0.0, q2)
                q3 = jnp.where(fl, 0.0, q3)
                x0 = jnp.where(fl, _NEG, x0)
                x1 = jnp.where(fl, _NEG, x1)
                x2 = jnp.where(fl, _NEG, x2)
                x3 = jnp.where(fl, _NEG, x3)
                n0 = jnp.where(fl, _POS, n0)
                n1 = jnp.where(fl, _POS, n1)
                n2 = jnp.where(fl, _POS, n2)
                n3 = jnp.where(fl, _POS, n3)
                return (s0, s1, s2, s3, q0, q1, q2, q3,
                        x0, x1, x2, x3, n0, n1, n2, n3)

            init = (zf, zf, zf, zf, zf, zf, zf, zf,
                    negv, negv, negv, negv, posv, posv, posv, posv)
            lax.fori_loop(0, _CH, _edge, init)
            return 0
        lax.fori_loop(0, nblk, _blk, 0)

    pltpu.sync_copy(accs, ssum_hbm.at[pl.ds(nbase, _NPW)])
    pltpu.sync_copy(accq, ssq_hbm.at[pl.ds(nbase, _NPW)])
    pltpu.sync_copy(accx, smx_hbm.at[pl.ds(nbase, _NPW)])
    pltpu.sync_copy(accn, smn_hbm.at[pl.ds(nbase, _NPW)])


def _aggregate(msg, seid, meta, tot16):
    return pl.kernel(
        _aggregate_body,
        out_type=(
            jax.ShapeDtypeStruct((_NP, _H), jnp.float32),
            jax.ShapeDtypeStruct((_NP, _H), jnp.float32),
            jax.ShapeDtypeStruct((_NP, _H), jnp.float32),
            jax.ShapeDtypeStruct((_NP, _H), jnp.float32),
        ),
        mesh=_mesh(),
        compiler_params=_SC_PARAMS,
        scratch_types=[
            pltpu.VMEM((_NPW, _H), jnp.float32),
            pltpu.VMEM((_NPW, _H), jnp.float32),
            pltpu.VMEM((_NPW, _H), jnp.float32),
            pltpu.VMEM((_NPW, _H), jnp.float32),
            pltpu.VMEM((_CAP2,), jnp.int32),
            pltpu.VMEM((_CAP2,), jnp.int32),
            pltpu.VMEM((2, _CH, _H), jnp.float32),
            pltpu.VMEM((16,), jnp.int32),
            pltpu.SemaphoreType.DMA((2,)),
        ],
    )(msg, seid, meta, tot16)


def _proj_body(h_ref, wd_ref, ws_ref, b1_ref, pd_ref, ps_ref):
    h = h_ref[...]
    pd_ref[...] = jnp.dot(h, wd_ref[...], preferred_element_type=jnp.float32) + b1_ref[...]
    ps_ref[...] = jnp.dot(h, ws_ref[...], preferred_element_type=jnp.float32)


def _proj(h, wd, ws, b1):
    return pl.pallas_call(
        _proj_body,
        out_shape=(jax.ShapeDtypeStruct((_NP, _H), jnp.float32),
                   jax.ShapeDtypeStruct((_NP, _H), jnp.float32)),
    )(h, wd, ws, b1.reshape(1, _H))


def _edge_mlp_body(g_ref, ea_ref, we_ref, w2_ref, b2_ref, out_ref):
    pre = g_ref[...] + jnp.dot(ea_ref[...], we_ref[...], preferred_element_type=jnp.float32)
    h = jnp.maximum(pre, 0.0)
    out_ref[...] = jnp.dot(h, w2_ref[...], preferred_element_type=jnp.float32) + b2_ref[...]


def _edge_mlp(g, ea, we, w2, b2):
    be = 1536
    grid = _EP // be
    return pl.pallas_call(
        _edge_mlp_body,
        out_shape=jax.ShapeDtypeStruct((_EP, _H), jnp.float32),
        grid=(grid,),
        in_specs=[
            pl.BlockSpec((be, _H), lambda i: (i, 0)),
            pl.BlockSpec((be, _H), lambda i: (i, 0)),
            pl.BlockSpec((_H, _H), lambda i: (0, 0)),
            pl.BlockSpec((_H, _H), lambda i: (0, 0)),
            pl.BlockSpec((1, _H), lambda i: (0, 0)),
        ],
        out_specs=pl.BlockSpec((be, _H), lambda i: (i, 0)),
    )(g, ea, we, w2, b2.reshape(1, _H))


def _update_body(ssum_ref, ssq_ref, smx_ref, smn_ref, cnt_ref, h_ref,
                 um_ref, un_ref, ux_ref, us_ref, uh_ref, ub1_ref,
                 w2_ref, ub2_ref, out_ref):
    cnt = cnt_ref[...]
    cntc = jnp.maximum(cnt, 1.0)
    mean = ssum_ref[...] / cntc
    msq = ssq_ref[...] / cntc
    std = jnp.sqrt(jnp.maximum(msq - mean * mean, 0.0) + 1e-5)
    pos = cnt > 0.0
    mx = jnp.where(pos, smx_ref[...], 0.0)
    mn = jnp.where(pos, smn_ref[...], 0.0)
    z = (jnp.dot(mean, um_ref[...], preferred_element_type=jnp.float32)
         + jnp.dot(mn, un_ref[...], preferred_element_type=jnp.float32)
         + jnp.dot(mx, ux_ref[...], preferred_element_type=jnp.float32)
         + jnp.dot(std, us_ref[...], preferred_element_type=jnp.float32)
         + jnp.dot(h_ref[...], uh_ref[...], preferred_element_type=jnp.float32)
         + ub1_ref[...])
    z = jnp.maximum(z, 0.0)
    out_ref[...] = jnp.dot(z, w2_ref[...], preferred_element_type=jnp.float32) + ub2_ref[...]


def _update(ssum, ssq, smx, smn, cnt2d, h, um, un, ux, us, uh, ub1, w2, ub2):
    return pl.pallas_call(
        _update_body,
        out_shape=jax.ShapeDtypeStruct((_NP, _H), jnp.float32),
    )(ssum, ssq, smx, smn, cnt2d, h, um, un, ux, us, uh,
      ub1.reshape(1, _H), w2, ub2.reshape(1, _H))


def _readout_body(h_ref, vb_ref, gf_ref, f1h_ref, f1g_ref, b1_ref,
                  w2_ref, b2_ref, w3_ref, b3_ref, out_ref):
    onehot = (vb_ref[...] == lax.broadcasted_iota(jnp.int32, (1, 16), 1).astype(jnp.float32)).astype(jnp.float32)
    g = jnp.dot(onehot, gf_ref[...], preferred_element_type=jnp.float32)
    y = (jnp.dot(h_ref[...], f1h_ref[...], preferred_element_type=jnp.float32)
         + jnp.dot(g, f1g_ref[...], preferred_element_type=jnp.float32)
         + b1_ref[...])
    y = jnp.maximum(y, 0.0)
    y = jnp.maximum(jnp.dot(y, w2_ref[...], preferred_element_type=jnp.float32) + b2_ref[...], 0.0)
    out_ref[...] = jnp.dot(y, w3_ref[...], preferred_element_type=jnp.float32) + b3_ref[...]


def _readout(h, vb16, gf, f1h, f1g, b1, w2, b2, w3p, b3p):
    return pl.pallas_call(
        _readout_body,
        out_shape=jax.ShapeDtypeStruct((_NP, 128), jnp.float32),
    )(h, vb16, gf, f1h, f1g, b1.reshape(1, _H), w2, b2.reshape(1, _H),
      w3p, b3p.reshape(1, 128))


def kernel(x, edge_index, edge_attr, global_features, vertex_batch_map,
           edge_batch_map, params):
    src = edge_index[0].astype(jnp.int32)
    dst = edge_index[1].astype(jnp.int32)

    peid, pdst, offs, cnt16 = _bucketize(dst)
    seid, meta, tot16, cntw = _sortlocal(peid, pdst, offs, cnt16)

    padidx = (jnp.arange(_EP - _E, dtype=jnp.int32) * 37) % _N
    dstp = jnp.concatenate([dst, padidx])
    srcp = jnp.concatenate([src, padidx])

    h = jnp.pad(x, ((0, _NP - _N), (0, 0)))
    ea = jnp.pad(edge_attr, ((0, _EP - _E), (0, _H - edge_attr.shape[1])))

    cnt_full = cntw[:, :_NPW].reshape(_NP)
    cnt2d = jnp.broadcast_to(cnt_full[:, None], (_NP, _H))

    for l in range(5):
        p = params
        pre = f'b{l}_'
        mW1 = p[pre + 'mW1']
        fdim = 128 if l == 0 else _H
        wd = mW1[:fdim]
        ws = mW1[fdim:2 * fdim]
        we = mW1[2 * fdim:]
        if l == 0:
            we = jnp.pad(we, ((0, _H - we.shape[0]), (0, 0)))
        pd, ps = _proj(h, wd, ws, p[pre + 'mb1'])
        g = _edge_gather(pd, ps, dstp, srcp)
        msg = _edge_mlp(g, ea, we, p[pre + 'mW2'], p[pre + 'mb2'])
        ssum, ssq, smx, smn = _aggregate(msg, seid, meta, tot16)
        uW1 = p[pre + 'uW1']
        um = uW1[0:_H]
        un = uW1[_H:2 * _H]
        ux = uW1[2 * _H:3 * _H]
        us = uW1[3 * _H:4 * _H]
        uh = uW1[4 * _H:]
        h = _update(ssum, ssq, smx, smn, cnt2d, h, um, un, ux, us, uh,
                    p[pre + 'ub1'], p[pre + 'uW2'], p[pre + 'ub2'])
        ea = msg

    vertex_embeddings = h[:_N]

    vbp = jnp.pad(vertex_batch_map.astype(jnp.float32), (0, _NP - _N))
    vb16 = jnp.broadcast_to(vbp[:, None], (_NP, 16))
    w3p = jnp.pad(params['fc3W'], ((0, 0), (0, 127)))
    b3p = jnp.pad(params['fc3b'], (0, 127))
    q = _readout(h, vb16, global_features, params['fc1W'][:_H],
                 params['fc1W'][_H:], params['fc1b'], params['fc2W'],
                 params['fc2b'], w3p, b3p)
    q_values = q[:_N, :1]
    return (vertex_embeddings, q_values)


# speedup vs baseline: 4.8106x; 1.3913x over previous
import jax
import jax.numpy as jnp
from jax import lax
from jax.experimental import pallas as pl
from jax.experimental.pallas import tpu as pltpu
from jax.experimental.pallas import tpu_sc as plsc

_N = 10000
_E = 320000
_H = 64
_NW = 32
_NPW = 313
_NP = _NW * _NPW
_EPW = _E // _NW
_EPAD = 11024
_BLK = 384
_EPW2 = 28 * _BLK
_EP = _NW * _EPW2
_CAP2 = 12288
_CH = 128
_NEG = -3.4e38
_POS = 3.4e38


def _wid():
    return lax.axis_index("s") * 2 + lax.axis_index("c")


def _mesh():
    return plsc.VectorSubcoreMesh(core_axis_name="c", subcore_axis_name="s")


_SC_PARAMS = pltpu.CompilerParams(needs_layout_passes=False, use_tc_tiling_on_sc=False)


def _bucketize_body(dst_hbm, peid_hbm, pdst_hbm, offs_hbm, cnt16_hbm,
                    stage, hist, cur, offsv, leid, ldst):
    w = _wid()
    base = w * _EPW
    lanes = lax.iota(jnp.int32, 16)
    zeros16 = jnp.zeros((16,), jnp.int32)
    ones16 = jnp.ones((16,), jnp.int32)

    def _zero_hist(i, _):
        hist[pl.ds(i * 16, 16)] = zeros16
        return 0
    lax.fori_loop(0, 32, _zero_hist, 0)

    def _zero_lists(i, _):
        leid[pl.ds(i * 16, 16)] = zeros16
        ldst[pl.ds(i * 16, 16)] = zeros16
        return 0
    lax.fori_loop(0, _EPAD // 16, _zero_lists, 0)

    def _chunk1(c, _):
        pltpu.sync_copy(dst_hbm.at[pl.ds(base + c * 2000, 2000)], stage)
        def _vreg(k, _):
            v = stage[pl.ds(k * 16, 16)]
            bkt = lax.div(v, _NPW)
            plsc.addupdate_scatter(hist, [bkt * 16 + lanes], ones16)
            return 0
        lax.fori_loop(0, 125, _vreg, 0)
        return 0
    lax.fori_loop(0, 5, _chunk1, 0)

    carry = jnp.int32(0)
    for b in range(32):
        hv = hist[pl.ds(b * 16, 16)]
        cs = plsc.cumsum(hv)
        cur[pl.ds(b * 16, 16)] = cs - hv + carry
        plsc.store_scatter(offsv, [jnp.full((16,), b, jnp.int32)],
                           jnp.full((16,), 1, jnp.int32) * carry,
                           mask=lanes == 0)
        carry = lax.div(carry + cs[15] + 15, 16) * 16

    def _chunk2(c, _):
        pltpu.sync_copy(dst_hbm.at[pl.ds(base + c * 2000, 2000)], stage)
        def _vreg(k, _):
            v = stage[pl.ds(k * 16, 16)]
            bkt = lax.div(v, _NPW)
            key = bkt * 16 + lanes
            pos = plsc.load_gather(cur, [key])
            eid = jnp.full((16,), base + c * 2000 + k * 16, jnp.int32) + lanes
            plsc.store_scatter(leid, [pos], eid)
            plsc.store_scatter(ldst, [pos], v)
            plsc.store_scatter(cur, [key], pos + 1)
            return 0
        lax.fori_loop(0, 125, _vreg, 0)
        return 0
    lax.fori_loop(0, 5, _chunk2, 0)

    pltpu.sync_copy(leid, peid_hbm.at[w])
    pltpu.sync_copy(ldst, pdst_hbm.at[w])
    pltpu.sync_copy(offsv, offs_hbm.at[w])
    pltpu.sync_copy(hist, cnt16_hbm.at[w])


def _bucketize(dst):
    return pl.kernel(
        _bucketize_body,
        out_type=(
            jax.ShapeDtypeStruct((_NW, _EPAD), jnp.int32),
            jax.ShapeDtypeStruct((_NW, _EPAD), jnp.int32),
            jax.ShapeDtypeStruct((_NW, 32), jnp.int32),
            jax.ShapeDtypeStruct((_NW, 512), jnp.int32),
        ),
        mesh=_mesh(),
        compiler_params=_SC_PARAMS,
        scratch_types=[
            pltpu.VMEM((2000,), jnp.int32),
            pltpu.VMEM((512,), jnp.int32),
            pltpu.VMEM((512,), jnp.int32),
            pltpu.VMEM((32,), jnp.int32),
            pltpu.VMEM((_EPAD,), jnp.int32),
            pltpu.VMEM((_EPAD,), jnp.int32),
        ],
    )(dst)


def _sortlocal_body(peid_hbm, pdst_hbm, offs_hbm, cnt16_hbm,
                    seid_hbm, meta_hbm, tot16_hbm, cntw_hbm,
                    eidc, dstc, hist, cur, cntf, seidl, sdstl, offsv, c16, t16):
    b = _wid()
    nbase = b * _NPW
    lanes = lax.iota(jnp.int32, 16)
    zeros16 = jnp.zeros((16,), jnp.int32)
    ones16 = jnp.ones((16,), jnp.int32)

    def _zh(i, _):
        hist[pl.ds(i * 16, 16)] = zeros16
        return 0
    lax.fori_loop(0, _NPW, _zh, 0)

    def _zs(i, _):
        seidl[pl.ds(i * 16, 16)] = zeros16
        sdstl[pl.ds(i * 16, 16)] = zeros16
        return 0
    lax.fori_loop(0, (_CAP2 + 16) // 16, _zs, 0)

    for w in range(_NW):
        pltpu.sync_copy(offs_hbm.at[w], offsv)
        start = pl.multiple_of(
            plsc.load_gather(offsv, [jnp.full((16,), b, jnp.int32)])[0], 16)
        pltpu.sync_copy(cnt16_hbm.at[w].at[pl.ds(pl.multiple_of(b * 16, 16), 16)], c16)
        cnt_wb = jnp.sum(c16[...])
        nch = lax.div(cnt_wb + 511, 512)

        def _chunk(c, _):
            off = pl.multiple_of(start + c * 512, 16)
            pltpu.sync_copy(pdst_hbm.at[w].at[pl.ds(off, 512)], dstc)
            def _vreg(k, _):
                dv = dstc[pl.ds(k * 16, 16)]
                tv = jnp.clip(dv - nbase, 0, _NPW - 1)
                inr = (c * 512 + k * 16 + lanes) < cnt_wb
                plsc.addupdate_scatter(hist, [tv * 16 + lanes], ones16, mask=inr)
                return 0
            lax.fori_loop(0, 32, _vreg, 0)
            return 0
        lax.fori_loop(0, nch, _chunk, 0)

    def _scan(i, carry):
        hv = hist[pl.ds(i * 16, 16)]
        cs = plsc.cumsum(hv)
        cur[pl.ds(i * 16, 16)] = cs - hv + carry
        plsc.store_scatter(cntf, [jnp.full((16,), 0, jnp.int32) + i],
                           jnp.zeros((16,), jnp.float32) + cs[15].astype(jnp.float32),
                           mask=lanes == 0)
        return carry + cs[15]
    total = lax.fori_loop(0, _NPW, _scan, jnp.int32(0))

    for w in range(_NW):
        pltpu.sync_copy(offs_hbm.at[w], offsv)
        start = pl.multiple_of(
            plsc.load_gather(offsv, [jnp.full((16,), b, jnp.int32)])[0], 16)
        pltpu.sync_copy(cnt16_hbm.at[w].at[pl.ds(pl.multiple_of(b * 16, 16), 16)], c16)
        cnt_wb = jnp.sum(c16[...])
        nch = lax.div(cnt_wb + 511, 512)

        def _chunk(c, _):
            off = pl.multiple_of(start + c * 512, 16)
            pltpu.sync_copy(peid_hbm.at[w].at[pl.ds(off, 512)], eidc)
            pltpu.sync_copy(pdst_hbm.at[w].at[pl.ds(off, 512)], dstc)
            def _vreg(k, _):
                dv = dstc[pl.ds(k * 16, 16)]
                ev = eidc[pl.ds(k * 16, 16)]
                tv = jnp.clip(dv - nbase, 0, _NPW - 1)
                key = tv * 16 + lanes
                inr = (c * 512 + k * 16 + lanes) < cnt_wb
                pos = plsc.load_gather(cur, [key])
                plsc.store_scatter(seidl, [pos], ev, mask=inr)
                plsc.store_scatter(sdstl, [pos], dv, mask=inr)
                plsc.store_scatter(cur, [key], pos + 1, mask=inr)
                return 0
            lax.fori_loop(0, 32, _vreg, 0)
            return 0
        lax.fori_loop(0, nch, _chunk, 0)

    def _meta(i, _):
        dv = sdstl[pl.ds(i * 16, 16)]
        dn = plsc.load_gather(sdstl, [i * 16 + 1 + lanes])
        tv = jnp.clip(dv - nbase, 0, _NPW - 1)
        fl = jnp.where(dv != dn, jnp.int32(1 << 16), jnp.int32(0))
        sdstl[pl.ds(i * 16, 16)] = tv + fl
        return 0
    lax.fori_loop(0, _CAP2 // 16, _meta, 0)

    t16[...] = jnp.full((16,), 1, jnp.int32) * total
    pltpu.sync_copy(seidl, seid_hbm.at[b])
    pltpu.sync_copy(sdstl.at[pl.ds(0, _CAP2)], meta_hbm.at[b])
    pltpu.sync_copy(t16, tot16_hbm.at[b])
    pltpu.sync_copy(cntf, cntw_hbm.at[b])


def _sortlocal(peid, pdst, offs, cnt16):
    return pl.kernel(
        _sortlocal_body,
        out_type=(
            jax.ShapeDtypeStruct((_NW, _CAP2), jnp.int32),
            jax.ShapeDtypeStruct((_NW, _CAP2), jnp.int32),
            jax.ShapeDtypeStruct((_NW, 16), jnp.int32),
            jax.ShapeDtypeStruct((_NW, 320), jnp.float32),
        ),
        mesh=_mesh(),
        compiler_params=_SC_PARAMS,
        scratch_types=[
            pltpu.VMEM((512,), jnp.int32),
            pltpu.VMEM((512,), jnp.int32),
            pltpu.VMEM((_NPW * 16,), jnp.int32),
            pltpu.VMEM((_NPW * 16,), jnp.int32),
            pltpu.VMEM((320,), jnp.float32),
            pltpu.VMEM((_CAP2,), jnp.int32),
            pltpu.VMEM((_CAP2 + 16,), jnp.int32),
            pltpu.VMEM((32,), jnp.int32),
            pltpu.VMEM((16,), jnp.int32),
            pltpu.VMEM((16,), jnp.int32),
        ],
    )(peid, pdst, offs, cnt16)


def _edge_gather_body(pd_hbm, ps_hbm, dst_hbm, src_hbm, g_hbm,
                      dstb, srcb, bufa, bufb, sema, semb):
    w = _wid()
    ebase = w * _EPW2
    nblk = _EPW2 // _BLK

    pltpu.sync_copy(dst_hbm.at[pl.ds(ebase, _EPW2)], dstb)
    pltpu.sync_copy(src_hbm.at[pl.ds(ebase, _EPW2)], srcb)

    def _issue(t, q):
        for k in range(_BLK // _CH):
            off = pl.multiple_of(t * _BLK + k * _CH, _CH)
            sl = pl.ds(k * _CH, _CH)
            pltpu.async_copy(pd_hbm.at[dstb.at[pl.ds(off, _CH)]],
                             bufa.at[q].at[sl], sema.at[q])
            pltpu.async_copy(ps_hbm.at[srcb.at[pl.ds(off, _CH)]],
                             bufb.at[q].at[sl], semb.at[q])

    def _drain(q):
        for k in range(_BLK // _CH):
            sl = pl.ds(k * _CH, _CH)
            pltpu.make_async_copy(pd_hbm.at[dstb.at[pl.ds(0, _CH)]],
                                  bufa.at[q].at[sl], sema.at[q]).wait()
            pltpu.make_async_copy(ps_hbm.at[srcb.at[pl.ds(0, _CH)]],
                                  bufb.at[q].at[sl], semb.at[q]).wait()

    _issue(0, 0)

    def _step(t, _):
        q = lax.rem(t, 2)

        @pl.when(t + 1 < nblk)
        def _():
            _issue(t + 1, 1 - q)

        _drain(q)

        def _row(r, _):
            for j in range(4):
                sl = pl.ds(j * 16, 16)
                bufa[q, r, sl] = bufa[q, r, sl] + bufb[q, r, sl]
            return 0
        lax.fori_loop(0, _BLK, _row, 0)
        pltpu.sync_copy(bufa.at[q], g_hbm.at[pl.ds(ebase + t * _BLK, _BLK)])
        return 0
    lax.fori_loop(0, nblk, _step, 0)


def _edge_gather(pd, ps, dst, src):
    return pl.kernel(
        _edge_gather_body,
        out_type=jax.ShapeDtypeStruct((_EP, _H), jnp.float32),
        mesh=_mesh(),
        compiler_params=_SC_PARAMS,
        scratch_types=[
            pltpu.VMEM((_EPW2,), jnp.int32),
            pltpu.VMEM((_EPW2,), jnp.int32),
            pltpu.VMEM((2, _BLK, _H), jnp.float32),
            pltpu.VMEM((2, _BLK, _H), jnp.float32),
            pltpu.SemaphoreType.DMA((2,)),
            pltpu.SemaphoreType.DMA((2,)),
        ],
    )(pd, ps, dst, src)


def _aggregate_body(msg_hbm, seid_hbm, meta_hbm, tot16_hbm,
                    ssum_hbm, ssq_hbm, smx_hbm, smn_hbm,
                    accs, accq, accx, accn, seidl, metal, gbuf, t16, sem):
    b = _wid()
    nbase = b * _NPW
    zf = jnp.zeros((16,), jnp.float32)
    negv = jnp.full((16,), _NEG, jnp.float32)
    posv = jnp.full((16,), _POS, jnp.float32)

    def _init(i, _):
        for j in range(4):
            sl = pl.ds(j * 16, 16)
            accs[i, sl] = zf
            accq[i, sl] = zf
            accx[i, sl] = negv
            accn[i, sl] = posv
        return 0
    lax.fori_loop(0, _NPW, _init, 0)

    pltpu.sync_copy(seid_hbm.at[b], seidl)
    pltpu.sync_copy(meta_hbm.at[b], metal)
    pltpu.sync_copy(tot16_hbm.at[b], t16)
    total = t16[...][0]
    nblk = lax.div(total + _CH - 1, _CH)

    def _issue(t, q):
        off = pl.multiple_of(t * _CH, _CH)
        pltpu.async_copy(msg_hbm.at[seidl.at[pl.ds(off, _CH)]], gbuf.at[q], sem.at[q])

    def _drain(q):
        pltpu.make_async_copy(msg_hbm.at[seidl.at[pl.ds(0, _CH)]],
                              gbuf.at[q], sem.at[q]).wait()

    @pl.when(nblk > 0)
    def _():
        _issue(0, 0)

        def _blk(t, _):
            q = lax.rem(t, 2)

            @pl.when(t + 1 < nblk)
            def _():
                _issue(t + 1, 1 - q)

            _drain(q)

            def _edge(e, regs):
                (s0, s1, s2, s3, q0, q1, q2, q3,
                 x0, x1, x2, x3, n0, n1, n2, n3) = regs
                idx = t * _CH + e
                mv = plsc.load_gather(metal, [jnp.full((16,), 0, jnp.int32) + idx])[0]
                tnode = jnp.minimum(mv & 0xFFFF, _NPW - 1)
                ok = idx < total
                fl = (mv >= (1 << 16)) | (e == _CH - 1)
                r0 = gbuf[q, e, pl.ds(0, 16)]
                r1 = gbuf[q, e, pl.ds(16, 16)]
                r2 = gbuf[q, e, pl.ds(32, 16)]
                r3 = gbuf[q, e, pl.ds(48, 16)]
                z0 = jnp.where(ok, r0, 0.0)
                z1 = jnp.where(ok, r1, 0.0)
                z2 = jnp.where(ok, r2, 0.0)
                z3 = jnp.where(ok, r3, 0.0)
                s0 = s0 + z0
                s1 = s1 + z1
                s2 = s2 + z2
                s3 = s3 + z3
                q0 = q0 + z0 * z0
                q1 = q1 + z1 * z1
                q2 = q2 + z2 * z2
                q3 = q3 + z3 * z3
                x0 = jnp.maximum(x0, jnp.where(ok, r0, _NEG))
                x1 = jnp.maximum(x1, jnp.where(ok, r1, _NEG))
                x2 = jnp.maximum(x2, jnp.where(ok, r2, _NEG))
                x3 = jnp.maximum(x3, jnp.where(ok, r3, _NEG))
                n0 = jnp.minimum(n0, jnp.where(ok, r0, _POS))
                n1 = jnp.minimum(n1, jnp.where(ok, r1, _POS))
                n2 = jnp.minimum(n2, jnp.where(ok, r2, _POS))
                n3 = jnp.minimum(n3, jnp.where(ok, r3, _POS))

                @pl.when(fl)
                def _():
                    svs = (s0, s1, s2, s3)
                    qvs = (q0, q1, q2, q3)
                    xvs = (x0, x1, x2, x3)
                    nvs = (n0, n1, n2, n3)
                    for j in range(4):
                        sl = pl.ds(j * 16, 16)
                        accs[tnode, sl] = accs[tnode, sl] + svs[j]
                        accq[tnode, sl] = accq[tnode, sl] + qvs[j]
                        accx[tnode, sl] = jnp.maximum(accx[tnode, sl], xvs[j])
                        accn[tnode, sl] = jnp.minimum(accn[tnode, sl], nvs[j])

                s0 = jnp.where(fl, 0.0, s0)
                s1 = jnp.where(fl, 0.0, s1)
                s2 = jnp.where(fl, 0.0, s2)
                s3 = jnp.where(fl, 0.0, s3)
                q0 = jnp.where(fl, 0.0, q0)
                q1 = jnp.where(fl, 0.0, q1)
                q2 = jnp.where(fl, 0.0, q2)
                q3 = jnp.where(fl, 0.0, q3)
                x0 = jnp.where(fl, _NEG, x0)
                x1 = jnp.where(fl, _NEG, x1)
                x2 = jnp.where(fl, _NEG, x2)
                x3 = jnp.where(fl, _NEG, x3)
                n0 = jnp.where(fl, _POS, n0)
                n1 = jnp.where(fl, _POS, n1)
                n2 = jnp.where(fl, _POS, n2)
                n3 = jnp.where(fl, _POS, n3)
                return (s0, s1, s2, s3, q0, q1, q2, q3,
                        x0, x1, x2, x3, n0, n1, n2, n3)

            init = (zf, zf, zf, zf, zf, zf, zf, zf,
                    negv, negv, negv, negv, posv, posv, posv, posv)
            lax.fori_loop(0, _CH, _edge, init)
            return 0
        lax.fori_loop(0, nblk, _blk, 0)

    pltpu.sync_copy(accs, ssum_hbm.at[pl.ds(nbase, _NPW)])
    pltpu.sync_copy(accq, ssq_hbm.at[pl.ds(nbase, _NPW)])
    pltpu.sync_copy(accx, smx_hbm.at[pl.ds(nbase, _NPW)])
    pltpu.sync_copy(accn, smn_hbm.at[pl.ds(nbase, _NPW)])


def _aggregate(msg, seid, meta, tot16):
    return pl.kernel(
        _aggregate_body,
        out_type=(
            jax.ShapeDtypeStruct((_NP, _H), jnp.float32),
            jax.ShapeDtypeStruct((_NP, _H), jnp.float32),
            jax.ShapeDtypeStruct((_NP, _H), jnp.float32),
            jax.ShapeDtypeStruct((_NP, _H), jnp.float32),
        ),
        mesh=_mesh(),
        compiler_params=_SC_PARAMS,
        scratch_types=[
            pltpu.VMEM((_NPW, _H), jnp.float32),
            pltpu.VMEM((_NPW, _H), jnp.float32),
            pltpu.VMEM((_NPW, _H), jnp.float32),
            pltpu.VMEM((_NPW, _H), jnp.float32),
            pltpu.VMEM((_CAP2,), jnp.int32),
            pltpu.VMEM((_CAP2,), jnp.int32),
            pltpu.VMEM((2, _CH, _H), jnp.float32),
            pltpu.VMEM((16,), jnp.int32),
            pltpu.SemaphoreType.DMA((2,)),
        ],
    )(msg, seid, meta, tot16)


def _proj_body(h_ref, wd_ref, ws_ref, b1_ref, pd_ref, ps_ref):
    h = h_ref[...]
    pd_ref[...] = jnp.dot(h, wd_ref[...], preferred_element_type=jnp.float32) + b1_ref[...]
    ps_ref[...] = jnp.dot(h, ws_ref[...], preferred_element_type=jnp.float32)


def _proj(h, wd, ws, b1):
    return pl.pallas_call(
        _proj_body,
        out_shape=(jax.ShapeDtypeStruct((_NP, _H), jnp.float32),
                   jax.ShapeDtypeStruct((_NP, _H), jnp.float32)),
    )(h, wd, ws, b1.reshape(1, _H))


_BE2 = 768


def _edge_mlp0_body(g_ref, ea_ref, we_ref, w2_ref, b2_ref, out_ref):
    g2 = g_ref[...].reshape(_BE2, 128)
    pre = g2 + jnp.dot(ea_ref[...], we_ref[...], preferred_element_type=jnp.float32)
    h = jnp.maximum(pre, 0.0)
    m = jnp.dot(h, w2_ref[...], preferred_element_type=jnp.float32) + b2_ref[...]
    out_ref[...] = m.reshape(_BE2 * 128)


def _edge_mlp0(g1, ea2, we2, w22, b22):
    grid = _EP // (2 * _BE2)
    return pl.pallas_call(
        _edge_mlp0_body,
        out_shape=jax.ShapeDtypeStruct((_EP * _H,), jnp.float32),
        grid=(grid,),
        in_specs=[
            pl.BlockSpec((_BE2 * 128,), lambda i: (i,)),
            pl.BlockSpec((_BE2, 32), lambda i: (i, 0)),
            pl.BlockSpec((32, 128), lambda i: (0, 0)),
            pl.BlockSpec((128, 128), lambda i: (0, 0)),
            pl.BlockSpec((1, 128), lambda i: (0, 0)),
        ],
        out_specs=pl.BlockSpec((_BE2 * 128,), lambda i: (i,)),
    )(g1, ea2, we2, w22, b22.reshape(1, 128))


def _edge_mlp1_body(g_ref, ea_ref, we_ref, w2_ref, b2_ref, out_ref):
    g2 = g_ref[...].reshape(_BE2, 128)
    ea2 = ea_ref[...].reshape(_BE2, 128)
    pre = g2 + jnp.dot(ea2, we_ref[...], preferred_element_type=jnp.float32)
    h = jnp.maximum(pre, 0.0)
    m = jnp.dot(h, w2_ref[...], preferred_element_type=jnp.float32) + b2_ref[...]
    out_ref[...] = m.reshape(_BE2 * 128)


def _edge_mlp1(g1, ea1, we2, w22, b22):
    grid = _EP // (2 * _BE2)
    return pl.pallas_call(
        _edge_mlp1_body,
        out_shape=jax.ShapeDtypeStruct((_EP * _H,), jnp.float32),
        grid=(grid,),
        in_specs=[
            pl.BlockSpec((_BE2 * 128,), lambda i: (i,)),
            pl.BlockSpec((_BE2 * 128,), lambda i: (i,)),
            pl.BlockSpec((128, 128), lambda i: (0, 0)),
            pl.BlockSpec((128, 128), lambda i: (0, 0)),
            pl.BlockSpec((1, 128), lambda i: (0, 0)),
        ],
        out_specs=pl.BlockSpec((_BE2 * 128,), lambda i: (i,)),
    )(g1, ea1, we2, w22, b22.reshape(1, 128))


def _update_body(ssum_ref, ssq_ref, smx_ref, smn_ref, cnt_ref, h_ref,
                 um_ref, un_ref, ux_ref, us_ref, uh_ref, ub1_ref,
                 w2_ref, ub2_ref, out_ref):
    cnt = cnt_ref[...]
    cntc = jnp.maximum(cnt, 1.0)
    mean = ssum_ref[...] / cntc
    msq = ssq_ref[...] / cntc
    std = jnp.sqrt(jnp.maximum(msq - mean * mean, 0.0) + 1e-5)
    pos = cnt > 0.0
    mx = jnp.where(pos, smx_ref[...], 0.0)
    mn = jnp.where(pos, smn_ref[...], 0.0)
    z = (jnp.dot(mean, um_ref[...], preferred_element_type=jnp.float32)
         + jnp.dot(mn, un_ref[...], preferred_element_type=jnp.float32)
         + jnp.dot(mx, ux_ref[...], preferred_element_type=jnp.float32)
         + jnp.dot(std, us_ref[...], preferred_element_type=jnp.float32)
         + jnp.dot(h_ref[...], uh_ref[...], preferred_element_type=jnp.float32)
         + ub1_ref[...])
    z = jnp.maximum(z, 0.0)
    out_ref[...] = jnp.dot(z, w2_ref[...], preferred_element_type=jnp.float32) + ub2_ref[...]


def _update(ssum, ssq, smx, smn, cnt2d, h, um, un, ux, us, uh, ub1, w2, ub2):
    return pl.pallas_call(
        _update_body,
        out_shape=jax.ShapeDtypeStruct((_NP, _H), jnp.float32),
    )(ssum, ssq, smx, smn, cnt2d, h, um, un, ux, us, uh,
      ub1.reshape(1, _H), w2, ub2.reshape(1, _H))


def _readout_body(h_ref, vb_ref, gf_ref, f1h_ref, f1g_ref, b1_ref,
                  w2_ref, b2_ref, w3_ref, b3_ref, out_ref):
    onehot = (vb_ref[...] == lax.broadcasted_iota(jnp.int32, (1, 16), 1).astype(jnp.float32)).astype(jnp.float32)
    g = jnp.dot(onehot, gf_ref[...], preferred_element_type=jnp.float32)
    y = (jnp.dot(h_ref[...], f1h_ref[...], preferred_element_type=jnp.float32)
         + jnp.dot(g, f1g_ref[...], preferred_element_type=jnp.float32)
         + b1_ref[...])
    y = jnp.maximum(y, 0.0)
    y = jnp.maximum(jnp.dot(y, w2_ref[...], preferred_element_type=jnp.float32) + b2_ref[...], 0.0)
    out_ref[...] = jnp.dot(y, w3_ref[...], preferred_element_type=jnp.float32) + b3_ref[...]


def _readout(h, vb16, gf, f1h, f1g, b1, w2, b2, w3p, b3p):
    return pl.pallas_call(
        _readout_body,
        out_shape=jax.ShapeDtypeStruct((_NP, 128), jnp.float32),
    )(h, vb16, gf, f1h, f1g, b1.reshape(1, _H), w2, b2.reshape(1, _H),
      w3p, b3p.reshape(1, 128))


def kernel(x, edge_index, edge_attr, global_features, vertex_batch_map,
           edge_batch_map, params):
    src = edge_index[0].astype(jnp.int32)
    dst = edge_index[1].astype(jnp.int32)

    peid, pdst, offs, cnt16 = _bucketize(dst)
    seid, meta, tot16, cntw = _sortlocal(peid, pdst, offs, cnt16)

    padidx = (jnp.arange(_EP - _E, dtype=jnp.int32) * 37) % _N
    dstp = jnp.concatenate([dst, padidx])
    srcp = jnp.concatenate([src, padidx])

    h = jnp.pad(x, ((0, _NP - _N), (0, 0)))
    ea2 = jnp.pad(edge_attr, ((0, _EP - _E), (0, 0))).reshape(_EP // 2, 32)
    ea1 = None

    cnt_full = cntw[:, :_NPW].reshape(_NP)
    cnt2d = jnp.broadcast_to(cnt_full[:, None], (_NP, _H))

    eye2 = jnp.eye(2, dtype=jnp.float32)
    for l in range(5):
        p = params
        pre = f'b{l}_'
        mW1 = p[pre + 'mW1']
        fdim = 128 if l == 0 else _H
        wd = mW1[:fdim]
        ws = mW1[fdim:2 * fdim]
        we = mW1[2 * fdim:]
        we2 = jnp.kron(eye2, we)
        w22 = jnp.kron(eye2, p[pre + 'mW2'])
        b22 = jnp.concatenate([p[pre + 'mb2'], p[pre + 'mb2']])
        pd, ps = _proj(h, wd, ws, p[pre + 'mb1'])
        g = _edge_gather(pd, ps, dstp, srcp)
        g1 = g.reshape(_EP * _H)
        if l == 0:
            msg1 = _edge_mlp0(g1, ea2, we2, w22, b22)
        else:
            msg1 = _edge_mlp1(g1, ea1, we2, w22, b22)
        msg = msg1.reshape(_EP, _H)
        ssum, ssq, smx, smn = _aggregate(msg, seid, meta, tot16)
        uW1 = p[pre + 'uW1']
        um = uW1[0:_H]
        un = uW1[_H:2 * _H]
        ux = uW1[2 * _H:3 * _H]
        us = uW1[3 * _H:4 * _H]
        uh = uW1[4 * _H:]
        h = _update(ssum, ssq, smx, smn, cnt2d, h, um, un, ux, us, uh,
                    p[pre + 'ub1'], p[pre + 'uW2'], p[pre + 'ub2'])
        ea1 = msg1

    vertex_embeddings = h[:_N]

    vbp = jnp.pad(vertex_batch_map.astype(jnp.float32), (0, _NP - _N))
    vb16 = jnp.broadcast_to(vbp[:, None], (_NP, 16))
    w3p = jnp.pad(params['fc3W'], ((0, 0), (0, 127)))
    b3p = jnp.pad(params['fc3b'], (0, 127))
    q = _readout(h, vb16, global_features, params['fc1W'][:_H],
                 params['fc1W'][_H:], params['fc1b'], params['fc2W'],
                 params['fc2b'], w3p, b3p)
    q_values = q[:_N, :1]
    return (vertex_embeddings, q_values)


# speedup vs baseline: 4.8189x; 1.0017x over previous
import jax
import jax.numpy as jnp
from jax import lax
from jax.experimental import pallas as pl
from jax.experimental.pallas import tpu as pltpu
from jax.experimental.pallas import tpu_sc as plsc

_N = 10000
_E = 320000
_H = 64
_NW = 32
_NPW = 313
_NP = _NW * _NPW
_EPW = _E // _NW
_EPAD = 11024
_BLK = 384
_EPW2 = 28 * _BLK
_EP = _NW * _EPW2
_CAP2 = 12288
_CH = 128
_NEG = -3.4e38
_POS = 3.4e38


def _wid():
    return lax.axis_index("s") * 2 + lax.axis_index("c")


def _mesh():
    return plsc.VectorSubcoreMesh(core_axis_name="c", subcore_axis_name="s")


_SC_PARAMS = pltpu.CompilerParams(needs_layout_passes=False, use_tc_tiling_on_sc=False)


def _bucketize_body(dst_hbm, peid_hbm, pdst_hbm, offs_hbm, cnt16_hbm,
                    stage, hist, cur, offsv, leid, ldst):
    w = _wid()
    base = w * _EPW
    lanes = lax.iota(jnp.int32, 16)
    zeros16 = jnp.zeros((16,), jnp.int32)
    ones16 = jnp.ones((16,), jnp.int32)

    def _zero_hist(i, _):
        hist[pl.ds(i * 16, 16)] = zeros16
        return 0
    lax.fori_loop(0, 32, _zero_hist, 0)

    def _zero_lists(i, _):
        leid[pl.ds(i * 16, 16)] = zeros16
        ldst[pl.ds(i * 16, 16)] = zeros16
        return 0
    lax.fori_loop(0, _EPAD // 16, _zero_lists, 0)

    def _chunk1(c, _):
        pltpu.sync_copy(dst_hbm.at[pl.ds(base + c * 2000, 2000)], stage)
        def _vreg(k, _):
            v = stage[pl.ds(k * 16, 16)]
            bkt = lax.div(v, _NPW)
            plsc.addupdate_scatter(hist, [bkt * 16 + lanes], ones16)
            return 0
        lax.fori_loop(0, 125, _vreg, 0)
        return 0
    lax.fori_loop(0, 5, _chunk1, 0)

    carry = jnp.int32(0)
    for b in range(32):
        hv = hist[pl.ds(b * 16, 16)]
        cs = plsc.cumsum(hv)
        cur[pl.ds(b * 16, 16)] = cs - hv + carry
        plsc.store_scatter(offsv, [jnp.full((16,), b, jnp.int32)],
                           jnp.full((16,), 1, jnp.int32) * carry,
                           mask=lanes == 0)
        carry = lax.div(carry + cs[15] + 15, 16) * 16

    def _chunk2(c, _):
        pltpu.sync_copy(dst_hbm.at[pl.ds(base + c * 2000, 2000)], stage)
        def _vreg(k, _):
            v = stage[pl.ds(k * 16, 16)]
            bkt = lax.div(v, _NPW)
            key = bkt * 16 + lanes
            pos = plsc.load_gather(cur, [key])
            eid = jnp.full((16,), base + c * 2000 + k * 16, jnp.int32) + lanes
            plsc.store_scatter(leid, [pos], eid)
            plsc.store_scatter(ldst, [pos], v)
            plsc.store_scatter(cur, [key], pos + 1)
            return 0
        lax.fori_loop(0, 125, _vreg, 0)
        return 0
    lax.fori_loop(0, 5, _chunk2, 0)

    pltpu.sync_copy(leid, peid_hbm.at[w])
    pltpu.sync_copy(ldst, pdst_hbm.at[w])
    pltpu.sync_copy(offsv, offs_hbm.at[w])
    pltpu.sync_copy(hist, cnt16_hbm.at[w])


def _bucketize(dst):
    return pl.kernel(
        _bucketize_body,
        out_type=(
            jax.ShapeDtypeStruct((_NW, _EPAD), jnp.int32),
            jax.ShapeDtypeStruct((_NW, _EPAD), jnp.int32),
            jax.ShapeDtypeStruct((_NW, 32), jnp.int32),
            jax.ShapeDtypeStruct((_NW, 512), jnp.int32),
        ),
        mesh=_mesh(),
        compiler_params=_SC_PARAMS,
        scratch_types=[
            pltpu.VMEM((2000,), jnp.int32),
            pltpu.VMEM((512,), jnp.int32),
            pltpu.VMEM((512,), jnp.int32),
            pltpu.VMEM((32,), jnp.int32),
            pltpu.VMEM((_EPAD,), jnp.int32),
            pltpu.VMEM((_EPAD,), jnp.int32),
        ],
    )(dst)


def _sortlocal_body(peid_hbm, pdst_hbm, offs_hbm, cnt16_hbm,
                    seid_hbm, meta_hbm, tot16_hbm, cntw_hbm,
                    eidc, dstc, hist, cur, cntf, seidl, sdstl, offsv, c16, t16):
    b = _wid()
    nbase = b * _NPW
    lanes = lax.iota(jnp.int32, 16)
    zeros16 = jnp.zeros((16,), jnp.int32)
    ones16 = jnp.ones((16,), jnp.int32)

    def _zh(i, _):
        hist[pl.ds(i * 16, 16)] = zeros16
        return 0
    lax.fori_loop(0, _NPW, _zh, 0)

    def _zs(i, _):
        seidl[pl.ds(i * 16, 16)] = zeros16
        sdstl[pl.ds(i * 16, 16)] = zeros16
        return 0
    lax.fori_loop(0, (_CAP2 + 16) // 16, _zs, 0)

    for w in range(_NW):
        pltpu.sync_copy(offs_hbm.at[w], offsv)
        start = pl.multiple_of(
            plsc.load_gather(offsv, [jnp.full((16,), b, jnp.int32)])[0], 16)
        pltpu.sync_copy(cnt16_hbm.at[w].at[pl.ds(pl.multiple_of(b * 16, 16), 16)], c16)
        cnt_wb = jnp.sum(c16[...])
        nch = lax.div(cnt_wb + 511, 512)

        def _chunk(c, _):
            off = pl.multiple_of(start + c * 512, 16)
            pltpu.sync_copy(pdst_hbm.at[w].at[pl.ds(off, 512)], dstc)
            def _vreg(k, _):
                dv = dstc[pl.ds(k * 16, 16)]
                tv = jnp.clip(dv - nbase, 0, _NPW - 1)
                inr = (c * 512 + k * 16 + lanes) < cnt_wb
                plsc.addupdate_scatter(hist, [tv * 16 + lanes], ones16, mask=inr)
                return 0
            lax.fori_loop(0, 32, _vreg, 0)
            return 0
        lax.fori_loop(0, nch, _chunk, 0)

    def _scan(i, carry):
        hv = hist[pl.ds(i * 16, 16)]
        cs = plsc.cumsum(hv)
        cur[pl.ds(i * 16, 16)] = cs - hv + carry
        plsc.store_scatter(cntf, [jnp.full((16,), 0, jnp.int32) + i],
                           jnp.zeros((16,), jnp.float32) + cs[15].astype(jnp.float32),
                           mask=lanes == 0)
        return carry + cs[15]
    total = lax.fori_loop(0, _NPW, _scan, jnp.int32(0))

    for w in range(_NW):
        pltpu.sync_copy(offs_hbm.at[w], offsv)
        start = pl.multiple_of(
            plsc.load_gather(offsv, [jnp.full((16,), b, jnp.int32)])[0], 16)
        pltpu.sync_copy(cnt16_hbm.at[w].at[pl.ds(pl.multiple_of(b * 16, 16), 16)], c16)
        cnt_wb = jnp.sum(c16[...])
        nch = lax.div(cnt_wb + 511, 512)

        def _chunk(c, _):
            off = pl.multiple_of(start + c * 512, 16)
            pltpu.sync_copy(peid_hbm.at[w].at[pl.ds(off, 512)], eidc)
            pltpu.sync_copy(pdst_hbm.at[w].at[pl.ds(off, 512)], dstc)
            def _vreg(k, _):
                dv = dstc[pl.ds(k * 16, 16)]
                ev = eidc[pl.ds(k * 16, 16)]
                tv = jnp.clip(dv - nbase, 0, _NPW - 1)
                key = tv * 16 + lanes
                inr = (c * 512 + k * 16 + lanes) < cnt_wb
                pos = plsc.load_gather(cur, [key])
                plsc.store_scatter(seidl, [pos], ev, mask=inr)
                plsc.store_scatter(sdstl, [pos], dv, mask=inr)
                plsc.store_scatter(cur, [key], pos + 1, mask=inr)
                return 0
            lax.fori_loop(0, 32, _vreg, 0)
            return 0
        lax.fori_loop(0, nch, _chunk, 0)

    def _meta(i, _):
        dv = sdstl[pl.ds(i * 16, 16)]
        dn = plsc.load_gather(sdstl, [i * 16 + 1 + lanes])
        tv = jnp.clip(dv - nbase, 0, _NPW - 1)
        fl = jnp.where(dv != dn, jnp.int32(1 << 16), jnp.int32(0))
        sdstl[pl.ds(i * 16, 16)] = tv + fl
        return 0
    lax.fori_loop(0, _CAP2 // 16, _meta, 0)

    t16[...] = jnp.full((16,), 1, jnp.int32) * total
    pltpu.sync_copy(seidl, seid_hbm.at[b])
    pltpu.sync_copy(sdstl.at[pl.ds(0, _CAP2)], meta_hbm.at[b])
    pltpu.sync_copy(t16, tot16_hbm.at[b])
    pltpu.sync_copy(cntf, cntw_hbm.at[b])


def _sortlocal(peid, pdst, offs, cnt16):
    return pl.kernel(
        _sortlocal_body,
        out_type=(
            jax.ShapeDtypeStruct((_NW, _CAP2), jnp.int32),
            jax.ShapeDtypeStruct((_NW, _CAP2), jnp.int32),
            jax.ShapeDtypeStruct((_NW, 16), jnp.int32),
            jax.ShapeDtypeStruct((_NW, 320), jnp.float32),
        ),
        mesh=_mesh(),
        compiler_params=_SC_PARAMS,
        scratch_types=[
            pltpu.VMEM((512,), jnp.int32),
            pltpu.VMEM((512,), jnp.int32),
            pltpu.VMEM((_NPW * 16,), jnp.int32),
            pltpu.VMEM((_NPW * 16,), jnp.int32),
            pltpu.VMEM((320,), jnp.float32),
            pltpu.VMEM((_CAP2,), jnp.int32),
            pltpu.VMEM((_CAP2 + 16,), jnp.int32),
            pltpu.VMEM((32,), jnp.int32),
            pltpu.VMEM((16,), jnp.int32),
            pltpu.VMEM((16,), jnp.int32),
        ],
    )(peid, pdst, offs, cnt16)


def _edge_gather_body(pd_hbm, ps_hbm, dst_hbm, src_hbm, g_hbm,
                      dstb, srcb, bufa, bufb, sema, semb, semw):
    w = _wid()
    ebase = w * _EPW2
    nblk = _EPW2 // _BLK

    pltpu.sync_copy(dst_hbm.at[pl.ds(ebase, _EPW2)], dstb)
    pltpu.sync_copy(src_hbm.at[pl.ds(ebase, _EPW2)], srcb)

    def _issue(t, q):
        for k in range(_BLK // _CH):
            off = pl.multiple_of(t * _BLK + k * _CH, _CH)
            sl = pl.ds(k * _CH, _CH)
            pltpu.async_copy(pd_hbm.at[dstb.at[pl.ds(off, _CH)]],
                             bufa.at[q].at[sl], sema.at[q])
            pltpu.async_copy(ps_hbm.at[srcb.at[pl.ds(off, _CH)]],
                             bufb.at[q].at[sl], semb.at[q])

    def _drain(q):
        for k in range(_BLK // _CH):
            sl = pl.ds(k * _CH, _CH)
            pltpu.make_async_copy(pd_hbm.at[dstb.at[pl.ds(0, _CH)]],
                                  bufa.at[q].at[sl], sema.at[q]).wait()
            pltpu.make_async_copy(ps_hbm.at[srcb.at[pl.ds(0, _CH)]],
                                  bufb.at[q].at[sl], semb.at[q]).wait()

    _issue(0, 0)

    def _step(t, _):
        q = lax.rem(t, 2)

        @pl.when(t + 1 < nblk)
        def _():
            @pl.when(t >= 1)
            def _():
                pltpu.make_async_copy(bufa.at[1 - q], g_hbm.at[pl.ds(ebase, _BLK)],
                                      semw.at[1 - q]).wait()
            _issue(t + 1, 1 - q)

        _drain(q)

        def _row(r, _):
            for j in range(4):
                sl = pl.ds(j * 16, 16)
                bufa[q, r, sl] = bufa[q, r, sl] + bufb[q, r, sl]
            return 0
        lax.fori_loop(0, _BLK, _row, 0)
        pltpu.async_copy(bufa.at[q], g_hbm.at[pl.ds(ebase + t * _BLK, _BLK)],
                         semw.at[q])
        return 0
    lax.fori_loop(0, nblk, _step, 0)
    pltpu.make_async_copy(bufa.at[0], g_hbm.at[pl.ds(ebase, _BLK)], semw.at[0]).wait()
    pltpu.make_async_copy(bufa.at[1], g_hbm.at[pl.ds(ebase, _BLK)], semw.at[1]).wait()


def _edge_gather(pd, ps, dst, src):
    return pl.kernel(
        _edge_gather_body,
        out_type=jax.ShapeDtypeStruct((_EP, _H), jnp.float32),
        mesh=_mesh(),
        compiler_params=_SC_PARAMS,
        scratch_types=[
            pltpu.VMEM((_EPW2,), jnp.int32),
            pltpu.VMEM((_EPW2,), jnp.int32),
            pltpu.VMEM((2, _BLK, _H), jnp.float32),
            pltpu.VMEM((2, _BLK, _H), jnp.float32),
            pltpu.SemaphoreType.DMA((2,)),
            pltpu.SemaphoreType.DMA((2,)),
            pltpu.SemaphoreType.DMA((2,)),
        ],
    )(pd, ps, dst, src)


def _aggregate_body(msg_hbm, seid_hbm, meta_hbm, tot16_hbm,
                    ssum_hbm, ssq_hbm, smx_hbm, smn_hbm,
                    accs, accq, accx, accn, seidl, metal, gbuf, t16, sem):
    b = _wid()
    nbase = b * _NPW
    zf = jnp.zeros((16,), jnp.float32)
    negv = jnp.full((16,), _NEG, jnp.float32)
    posv = jnp.full((16,), _POS, jnp.float32)

    def _init(i, _):
        for j in range(4):
            sl = pl.ds(j * 16, 16)
            accs[i, sl] = zf
            accq[i, sl] = zf
            accx[i, sl] = negv
            accn[i, sl] = posv
        return 0
    lax.fori_loop(0, _NPW, _init, 0)

    pltpu.sync_copy(seid_hbm.at[b], seidl)
    pltpu.sync_copy(meta_hbm.at[b], metal)
    pltpu.sync_copy(tot16_hbm.at[b], t16)
    total = t16[...][0]
    nblk = lax.div(total + _CH - 1, _CH)

    def _issue(t, q):
        off = pl.multiple_of(t * _CH, _CH)
        pltpu.async_copy(msg_hbm.at[seidl.at[pl.ds(off, _CH)]], gbuf.at[q], sem.at[q])

    def _drain(q):
        pltpu.make_async_copy(msg_hbm.at[seidl.at[pl.ds(0, _CH)]],
                              gbuf.at[q], sem.at[q]).wait()

    @pl.when(nblk > 0)
    def _():
        _issue(0, 0)

        def _blk(t, _):
            q = lax.rem(t, 2)

            @pl.when(t + 1 < nblk)
            def _():
                _issue(t + 1, 1 - q)

            _drain(q)

            def _edge_any(e, regs, masked):
                (s0, s1, s2, s3, q0, q1, q2, q3,
                 x0, x1, x2, x3, n0, n1, n2, n3) = regs
                idx = t * _CH + e
                mv = plsc.load_gather(metal, [jnp.full((16,), 0, jnp.int32) + idx])[0]
                tnode = jnp.minimum(mv & 0xFFFF, _NPW - 1)
                fl = (mv >= (1 << 16)) | (e == _CH - 1)
                r0 = gbuf[q, e, pl.ds(0, 16)]
                r1 = gbuf[q, e, pl.ds(16, 16)]
                r2 = gbuf[q, e, pl.ds(32, 16)]
                r3 = gbuf[q, e, pl.ds(48, 16)]
                if masked:
                    ok = idx < total
                    z0 = jnp.where(ok, r0, 0.0)
                    z1 = jnp.where(ok, r1, 0.0)
                    z2 = jnp.where(ok, r2, 0.0)
                    z3 = jnp.where(ok, r3, 0.0)
                    m0 = jnp.where(ok, r0, _NEG)
                    m1 = jnp.where(ok, r1, _NEG)
                    m2 = jnp.where(ok, r2, _NEG)
                    m3 = jnp.where(ok, r3, _NEG)
                    p0 = jnp.where(ok, r0, _POS)
                    p1 = jnp.where(ok, r1, _POS)
                    p2 = jnp.where(ok, r2, _POS)
                    p3 = jnp.where(ok, r3, _POS)
                else:
                    z0, z1, z2, z3 = r0, r1, r2, r3
                    m0, m1, m2, m3 = r0, r1, r2, r3
                    p0, p1, p2, p3 = r0, r1, r2, r3
                s0 = s0 + z0
                s1 = s1 + z1
                s2 = s2 + z2
                s3 = s3 + z3
                q0 = q0 + z0 * z0
                q1 = q1 + z1 * z1
                q2 = q2 + z2 * z2
                q3 = q3 + z3 * z3
                x0 = jnp.maximum(x0, m0)
                x1 = jnp.maximum(x1, m1)
                x2 = jnp.maximum(x2, m2)
                x3 = jnp.maximum(x3, m3)
                n0 = jnp.minimum(n0, p0)
                n1 = jnp.minimum(n1, p1)
                n2 = jnp.minimum(n2, p2)
                n3 = jnp.minimum(n3, p3)

                @pl.when(fl)
                def _():
                    svs = (s0, s1, s2, s3)
                    qvs = (q0, q1, q2, q3)
                    xvs = (x0, x1, x2, x3)
                    nvs = (n0, n1, n2, n3)
                    for j in range(4):
                        sl = pl.ds(j * 16, 16)
                        accs[tnode, sl] = accs[tnode, sl] + svs[j]
                        accq[tnode, sl] = accq[tnode, sl] + qvs[j]
                        accx[tnode, sl] = jnp.maximum(accx[tnode, sl], xvs[j])
                        accn[tnode, sl] = jnp.minimum(accn[tnode, sl], nvs[j])

                s0 = jnp.where(fl, 0.0, s0)
                s1 = jnp.where(fl, 0.0, s1)
                s2 = jnp.where(fl, 0.0, s2)
                s3 = jnp.where(fl, 0.0, s3)
                q0 = jnp.where(fl, 0.0, q0)
                q1 = jnp.where(fl, 0.0, q1)
                q2 = jnp.where(fl, 0.0, q2)
                q3 = jnp.where(fl, 0.0, q3)
                x0 = jnp.where(fl, _NEG, x0)
                x1 = jnp.where(fl, _NEG, x1)
                x2 = jnp.where(fl, _NEG, x2)
                x3 = jnp.where(fl, _NEG, x3)
                n0 = jnp.where(fl, _POS, n0)
                n1 = jnp.where(fl, _POS, n1)
                n2 = jnp.where(fl, _POS, n2)
                n3 = jnp.where(fl, _POS, n3)
                return (s0, s1, s2, s3, q0, q1, q2, q3,
                        x0, x1, x2, x3, n0, n1, n2, n3)

            init = (zf, zf, zf, zf, zf, zf, zf, zf,
                    negv, negv, negv, negv, posv, posv, posv, posv)

            @pl.when(t + 1 < nblk)
            def _():
                lax.fori_loop(0, _CH, lambda e, r: _edge_any(e, r, False), init)

            @pl.when(t + 1 >= nblk)
            def _():
                lax.fori_loop(0, _CH, lambda e, r: _edge_any(e, r, True), init)
            return 0
        lax.fori_loop(0, nblk, _blk, 0)

    pltpu.sync_copy(accs, ssum_hbm.at[pl.ds(nbase, _NPW)])
    pltpu.sync_copy(accq, ssq_hbm.at[pl.ds(nbase, _NPW)])
    pltpu.sync_copy(accx, smx_hbm.at[pl.ds(nbase, _NPW)])
    pltpu.sync_copy(accn, smn_hbm.at[pl.ds(nbase, _NPW)])


def _aggregate(msg, seid, meta, tot16):
    return pl.kernel(
        _aggregate_body,
        out_type=(
            jax.ShapeDtypeStruct((_NP, _H), jnp.float32),
            jax.ShapeDtypeStruct((_NP, _H), jnp.float32),
            jax.ShapeDtypeStruct((_NP, _H), jnp.float32),
            jax.ShapeDtypeStruct((_NP, _H), jnp.float32),
        ),
        mesh=_mesh(),
        compiler_params=_SC_PARAMS,
        scratch_types=[
            pltpu.VMEM((_NPW, _H), jnp.float32),
            pltpu.VMEM((_NPW, _H), jnp.float32),
            pltpu.VMEM((_NPW, _H), jnp.float32),
            pltpu.VMEM((_NPW, _H), jnp.float32),
            pltpu.VMEM((_CAP2,), jnp.int32),
            pltpu.VMEM((_CAP2,), jnp.int32),
            pltpu.VMEM((2, _CH, _H), jnp.float32),
            pltpu.VMEM((16,), jnp.int32),
            pltpu.SemaphoreType.DMA((2,)),
        ],
    )(msg, seid, meta, tot16)


def _proj_body(h_ref, wd_ref, ws_ref, b1_ref, pd_ref, ps_ref):
    h = h_ref[...]
    pd_ref[...] = jnp.dot(h, wd_ref[...], preferred_element_type=jnp.float32) + b1_ref[...]
    ps_ref[...] = jnp.dot(h, ws_ref[...], preferred_element_type=jnp.float32)


def _proj(h, wd, ws, b1):
    return pl.pallas_call(
        _proj_body,
        out_shape=(jax.ShapeDtypeStruct((_NP, _H), jnp.float32),
                   jax.ShapeDtypeStruct((_NP, _H), jnp.float32)),
    )(h, wd, ws, b1.reshape(1, _H))


_BE2 = 768


def _edge_mlp0_body(g_ref, ea_ref, we_ref, w2_ref, b2_ref, out_ref):
    g2 = g_ref[...].reshape(_BE2, 128)
    pre = g2 + jnp.dot(ea_ref[...], we_ref[...], preferred_element_type=jnp.float32)
    h = jnp.maximum(pre, 0.0)
    m = jnp.dot(h, w2_ref[...], preferred_element_type=jnp.float32) + b2_ref[...]
    out_ref[...] = m.reshape(_BE2 * 128)


def _edge_mlp0(g1, ea2, we2, w22, b22):
    grid = _EP // (2 * _BE2)
    return pl.pallas_call(
        _edge_mlp0_body,
        out_shape=jax.ShapeDtypeStruct((_EP * _H,), jnp.float32),
        grid=(grid,),
        in_specs=[
            pl.BlockSpec((_BE2 * 128,), lambda i: (i,)),
            pl.BlockSpec((_BE2, 32), lambda i: (i, 0)),
            pl.BlockSpec((32, 128), lambda i: (0, 0)),
            pl.BlockSpec((128, 128), lambda i: (0, 0)),
            pl.BlockSpec((1, 128), lambda i: (0, 0)),
        ],
        out_specs=pl.BlockSpec((_BE2 * 128,), lambda i: (i,)),
    )(g1, ea2, we2, w22, b22.reshape(1, 128))


def _edge_mlp1_body(g_ref, ea_ref, we_ref, w2_ref, b2_ref, out_ref):
    g2 = g_ref[...].reshape(_BE2, 128)
    ea2 = ea_ref[...].reshape(_BE2, 128)
    pre = g2 + jnp.dot(ea2, we_ref[...], preferred_element_type=jnp.float32)
    h = jnp.maximum(pre, 0.0)
    m = jnp.dot(h, w2_ref[...], preferred_element_type=jnp.float32) + b2_ref[...]
    out_ref[...] = m.reshape(_BE2 * 128)


def _edge_mlp1(g1, ea1, we2, w22, b22):
    grid = _EP // (2 * _BE2)
    return pl.pallas_call(
        _edge_mlp1_body,
        out_shape=jax.ShapeDtypeStruct((_EP * _H,), jnp.float32),
        grid=(grid,),
        in_specs=[
            pl.BlockSpec((_BE2 * 128,), lambda i: (i,)),
            pl.BlockSpec((_BE2 * 128,), lambda i: (i,)),
            pl.BlockSpec((128, 128), lambda i: (0, 0)),
            pl.BlockSpec((128, 128), lambda i: (0, 0)),
            pl.BlockSpec((1, 128), lambda i: (0, 0)),
        ],
        out_specs=pl.BlockSpec((_BE2 * 128,), lambda i: (i,)),
    )(g1, ea1, we2, w22, b22.reshape(1, 128))


def _update_body(ssum_ref, ssq_ref, smx_ref, smn_ref, cnt_ref, h_ref,
                 um_ref, un_ref, ux_ref, us_ref, uh_ref, ub1_ref,
                 w2_ref, ub2_ref, out_ref):
    cnt = cnt_ref[...]
    cntc = jnp.maximum(cnt, 1.0)
    mean = ssum_ref[...] / cntc
    msq = ssq_ref[...] / cntc
    std = jnp.sqrt(jnp.maximum(msq - mean * mean, 0.0) + 1e-5)
    pos = cnt > 0.0
    mx = jnp.where(pos, smx_ref[...], 0.0)
    mn = jnp.where(pos, smn_ref[...], 0.0)
    z = (jnp.dot(mean, um_ref[...], preferred_element_type=jnp.float32)
         + jnp.dot(mn, un_ref[...], preferred_element_type=jnp.float32)
         + jnp.dot(mx, ux_ref[...], preferred_element_type=jnp.float32)
         + jnp.dot(std, us_ref[...], preferred_element_type=jnp.float32)
         + jnp.dot(h_ref[...], uh_ref[...], preferred_element_type=jnp.float32)
         + ub1_ref[...])
    z = jnp.maximum(z, 0.0)
    out_ref[...] = jnp.dot(z, w2_ref[...], preferred_element_type=jnp.float32) + ub2_ref[...]


def _update(ssum, ssq, smx, smn, cnt2d, h, um, un, ux, us, uh, ub1, w2, ub2):
    return pl.pallas_call(
        _update_body,
        out_shape=jax.ShapeDtypeStruct((_NP, _H), jnp.float32),
    )(ssum, ssq, smx, smn, cnt2d, h, um, un, ux, us, uh,
      ub1.reshape(1, _H), w2, ub2.reshape(1, _H))


def _readout_body(h_ref, vb_ref, gf_ref, f1h_ref, f1g_ref, b1_ref,
                  w2_ref, b2_ref, w3_ref, b3_ref, out_ref):
    onehot = (vb_ref[...] == lax.broadcasted_iota(jnp.int32, (1, 16), 1).astype(jnp.float32)).astype(jnp.float32)
    g = jnp.dot(onehot, gf_ref[...], preferred_element_type=jnp.float32)
    y = (jnp.dot(h_ref[...], f1h_ref[...], preferred_element_type=jnp.float32)
         + jnp.dot(g, f1g_ref[...], preferred_element_type=jnp.float32)
         + b1_ref[...])
    y = jnp.maximum(y, 0.0)
    y = jnp.maximum(jnp.dot(y, w2_ref[...], preferred_element_type=jnp.float32) + b2_ref[...], 0.0)
    out_ref[...] = jnp.dot(y, w3_ref[...], preferred_element_type=jnp.float32) + b3_ref[...]


def _readout(h, vb16, gf, f1h, f1g, b1, w2, b2, w3p, b3p):
    return pl.pallas_call(
        _readout_body,
        out_shape=jax.ShapeDtypeStruct((_NP, 128), jnp.float32),
    )(h, vb16, gf, f1h, f1g, b1.reshape(1, _H), w2, b2.reshape(1, _H),
      w3p, b3p.reshape(1, 128))


def kernel(x, edge_index, edge_attr, global_features, vertex_batch_map,
           edge_batch_map, params):
    src = edge_index[0].astype(jnp.int32)
    dst = edge_index[1].astype(jnp.int32)

    peid, pdst, offs, cnt16 = _bucketize(dst)
    seid, meta, tot16, cntw = _sortlocal(peid, pdst, offs, cnt16)

    padidx = (jnp.arange(_EP - _E, dtype=jnp.int32) * 37) % _N
    dstp = jnp.concatenate([dst, padidx])
    srcp = jnp.concatenate([src, padidx])

    h = jnp.pad(x, ((0, _NP - _N), (0, 0)))
    ea2 = jnp.pad(edge_attr, ((0, _EP - _E), (0, 0))).reshape(_EP // 2, 32)
    ea1 = None

    cnt_full = cntw[:, :_NPW].reshape(_NP)
    cnt2d = jnp.broadcast_to(cnt_full[:, None], (_NP, _H))

    eye2 = jnp.eye(2, dtype=jnp.float32)
    for l in range(5):
        p = params
        pre = f'b{l}_'
        mW1 = p[pre + 'mW1']
        fdim = 128 if l == 0 else _H
        wd = mW1[:fdim]
        ws = mW1[fdim:2 * fdim]
        we = mW1[2 * fdim:]
        we2 = jnp.kron(eye2, we)
        w22 = jnp.kron(eye2, p[pre + 'mW2'])
        b22 = jnp.concatenate([p[pre + 'mb2'], p[pre + 'mb2']])
        pd, ps = _proj(h, wd, ws, p[pre + 'mb1'])
        g = _edge_gather(pd, ps, dstp, srcp)
        g1 = g.reshape(_EP * _H)
        if l == 0:
            msg1 = _edge_mlp0(g1, ea2, we2, w22, b22)
        else:
            msg1 = _edge_mlp1(g1, ea1, we2, w22, b22)
        msg = msg1.reshape(_EP, _H)
        ssum, ssq, smx, smn = _aggregate(msg, seid, meta, tot16)
        uW1 = p[pre + 'uW1']
        um = uW1[0:_H]
        un = uW1[_H:2 * _H]
        ux = uW1[2 * _H:3 * _H]
        us = uW1[3 * _H:4 * _H]
        uh = uW1[4 * _H:]
        h = _update(ssum, ssq, smx, smn, cnt2d, h, um, un, ux, us, uh,
                    p[pre + 'ub1'], p[pre + 'uW2'], p[pre + 'ub2'])
        ea1 = msg1

    vertex_embeddings = h[:_N]

    vbp = jnp.pad(vertex_batch_map.astype(jnp.float32), (0, _NP - _N))
    vb16 = jnp.broadcast_to(vbp[:, None], (_NP, 16))
    w3p = jnp.pad(params['fc3W'], ((0, 0), (0, 127)))
    b3p = jnp.pad(params['fc3b'], (0, 127))
    q = _readout(h, vb16, global_features, params['fc1W'][:_H],
                 params['fc1W'][_H:], params['fc1b'], params['fc2W'],
                 params['fc2b'], w3p, b3p)
    q_values = q[:_N, :1]
    return (vertex_embeddings, q_values)


# speedup vs baseline: 4.8394x; 1.0043x over previous
import jax
import jax.numpy as jnp
from jax import lax
from jax.experimental import pallas as pl
from jax.experimental.pallas import tpu as pltpu
from jax.experimental.pallas import tpu_sc as plsc

_N = 10000
_E = 320000
_H = 64
_NW = 32
_NPW = 313
_NP = _NW * _NPW
_EPW = _E // _NW
_EPAD = 11024
_BLK = 256
_EPW2 = 42 * _BLK
_EP = _NW * _EPW2
_CAP2 = 11264
_CH = 128
_NEG = -3.4e38
_POS = 3.4e38


def _wid():
    return lax.axis_index("s") * 2 + lax.axis_index("c")


def _mesh():
    return plsc.VectorSubcoreMesh(core_axis_name="c", subcore_axis_name="s")


_SC_PARAMS = pltpu.CompilerParams(needs_layout_passes=False, use_tc_tiling_on_sc=False)


def _bucketize_body(dst_hbm, peid_hbm, pdst_hbm, offs_hbm, cnt16_hbm,
                    stage, hist, cur, offsv, leid, ldst):
    w = _wid()
    base = w * _EPW
    lanes = lax.iota(jnp.int32, 16)
    zeros16 = jnp.zeros((16,), jnp.int32)
    ones16 = jnp.ones((16,), jnp.int32)

    def _zero_hist(i, _):
        hist[pl.ds(i * 16, 16)] = zeros16
        return 0
    lax.fori_loop(0, 32, _zero_hist, 0)

    def _zero_lists(i, _):
        leid[pl.ds(i * 16, 16)] = zeros16
        ldst[pl.ds(i * 16, 16)] = zeros16
        return 0
    lax.fori_loop(0, _EPAD // 16, _zero_lists, 0)

    def _chunk1(c, _):
        pltpu.sync_copy(dst_hbm.at[pl.ds(base + c * 2000, 2000)], stage)
        def _vreg(k, _):
            v = stage[pl.ds(k * 16, 16)]
            bkt = lax.div(v, _NPW)
            plsc.addupdate_scatter(hist, [bkt * 16 + lanes], ones16)
            return 0
        lax.fori_loop(0, 125, _vreg, 0)
        return 0
    lax.fori_loop(0, 5, _chunk1, 0)

    carry = jnp.int32(0)
    for b in range(32):
        hv = hist[pl.ds(b * 16, 16)]
        cs = plsc.cumsum(hv)
        cur[pl.ds(b * 16, 16)] = cs - hv + carry
        plsc.store_scatter(offsv, [jnp.full((16,), b, jnp.int32)],
                           jnp.full((16,), 1, jnp.int32) * carry,
                           mask=lanes == 0)
        carry = lax.div(carry + cs[15] + 15, 16) * 16

    def _chunk2(c, _):
        pltpu.sync_copy(dst_hbm.at[pl.ds(base + c * 2000, 2000)], stage)
        def _vreg(k, _):
            v = stage[pl.ds(k * 16, 16)]
            bkt = lax.div(v, _NPW)
            key = bkt * 16 + lanes
            pos = plsc.load_gather(cur, [key])
            eid = jnp.full((16,), base + c * 2000 + k * 16, jnp.int32) + lanes
            plsc.store_scatter(leid, [pos], eid)
            plsc.store_scatter(ldst, [pos], v)
            plsc.store_scatter(cur, [key], pos + 1)
            return 0
        lax.fori_loop(0, 125, _vreg, 0)
        return 0
    lax.fori_loop(0, 5, _chunk2, 0)

    pltpu.sync_copy(leid, peid_hbm.at[w])
    pltpu.sync_copy(ldst, pdst_hbm.at[w])
    pltpu.sync_copy(offsv, offs_hbm.at[w])
    pltpu.sync_copy(hist, cnt16_hbm.at[w])


def _bucketize(dst):
    return pl.kernel(
        _bucketize_body,
        out_type=(
            jax.ShapeDtypeStruct((_NW, _EPAD), jnp.int32),
            jax.ShapeDtypeStruct((_NW, _EPAD), jnp.int32),
            jax.ShapeDtypeStruct((_NW, 32), jnp.int32),
            jax.ShapeDtypeStruct((_NW, 512), jnp.int32),
        ),
        mesh=_mesh(),
        compiler_params=_SC_PARAMS,
        scratch_types=[
            pltpu.VMEM((2000,), jnp.int32),
            pltpu.VMEM((512,), jnp.int32),
            pltpu.VMEM((512,), jnp.int32),
            pltpu.VMEM((32,), jnp.int32),
            pltpu.VMEM((_EPAD,), jnp.int32),
            pltpu.VMEM((_EPAD,), jnp.int32),
        ],
    )(dst)


def _sortlocal_body(peid_hbm, pdst_hbm, offs_hbm, cnt16_hbm,
                    seid_hbm, meta_hbm, tot16_hbm, cntw_hbm,
                    eidc, dstc, hist, cur, cntf, seidl, sdstl, offsv, c16, t16):
    b = _wid()
    nbase = b * _NPW
    lanes = lax.iota(jnp.int32, 16)
    zeros16 = jnp.zeros((16,), jnp.int32)
    ones16 = jnp.ones((16,), jnp.int32)

    def _zh(i, _):
        hist[pl.ds(i * 16, 16)] = zeros16
        return 0
    lax.fori_loop(0, _NPW, _zh, 0)

    def _zs(i, _):
        seidl[pl.ds(i * 16, 16)] = zeros16
        sdstl[pl.ds(i * 16, 16)] = zeros16
        return 0
    lax.fori_loop(0, (_CAP2 + 16) // 16, _zs, 0)

    for w in range(_NW):
        pltpu.sync_copy(offs_hbm.at[w], offsv)
        start = pl.multiple_of(
            plsc.load_gather(offsv, [jnp.full((16,), b, jnp.int32)])[0], 16)
        pltpu.sync_copy(cnt16_hbm.at[w].at[pl.ds(pl.multiple_of(b * 16, 16), 16)], c16)
        cnt_wb = jnp.sum(c16[...])
        nch = lax.div(cnt_wb + 511, 512)

        def _chunk(c, _):
            off = pl.multiple_of(start + c * 512, 16)
            pltpu.sync_copy(pdst_hbm.at[w].at[pl.ds(off, 512)], dstc)
            def _vreg(k, _):
                dv = dstc[pl.ds(k * 16, 16)]
                tv = jnp.clip(dv - nbase, 0, _NPW - 1)
                inr = (c * 512 + k * 16 + lanes) < cnt_wb
                plsc.addupdate_scatter(hist, [tv * 16 + lanes], ones16, mask=inr)
                return 0
            lax.fori_loop(0, 32, _vreg, 0)
            return 0
        lax.fori_loop(0, nch, _chunk, 0)

    def _scan(i, carry):
        hv = hist[pl.ds(i * 16, 16)]
        cs = plsc.cumsum(hv)
        cur[pl.ds(i * 16, 16)] = cs - hv + carry
        plsc.store_scatter(cntf, [jnp.full((16,), 0, jnp.int32) + i],
                           jnp.zeros((16,), jnp.float32) + cs[15].astype(jnp.float32),
                           mask=lanes == 0)
        return carry + cs[15]
    total = lax.fori_loop(0, _NPW, _scan, jnp.int32(0))

    for w in range(_NW):
        pltpu.sync_copy(offs_hbm.at[w], offsv)
        start = pl.multiple_of(
            plsc.load_gather(offsv, [jnp.full((16,), b, jnp.int32)])[0], 16)
        pltpu.sync_copy(cnt16_hbm.at[w].at[pl.ds(pl.multiple_of(b * 16, 16), 16)], c16)
        cnt_wb = jnp.sum(c16[...])
        nch = lax.div(cnt_wb + 511, 512)

        def _chunk(c, _):
            off = pl.multiple_of(start + c * 512, 16)
            pltpu.sync_copy(peid_hbm.at[w].at[pl.ds(off, 512)], eidc)
            pltpu.sync_copy(pdst_hbm.at[w].at[pl.ds(off, 512)], dstc)
            def _vreg(k, _):
                dv = dstc[pl.ds(k * 16, 16)]
                ev = eidc[pl.ds(k * 16, 16)]
                tv = jnp.clip(dv - nbase, 0, _NPW - 1)
                key = tv * 16 + lanes
                inr = (c * 512 + k * 16 + lanes) < cnt_wb
                pos = plsc.load_gather(cur, [key])
                plsc.store_scatter(seidl, [pos], ev, mask=inr)
                plsc.store_scatter(sdstl, [pos], dv, mask=inr)
                plsc.store_scatter(cur, [key], pos + 1, mask=inr)
                return 0
            lax.fori_loop(0, 32, _vreg, 0)
            return 0
        lax.fori_loop(0, nch, _chunk, 0)

    def _meta(i, _):
        dv = sdstl[pl.ds(i * 16, 16)]
        dn = plsc.load_gather(sdstl, [i * 16 + 1 + lanes])
        tv = jnp.clip(dv - nbase, 0, _NPW - 1)
        fl = jnp.where(dv != dn, jnp.int32(1 << 16), jnp.int32(0))
        sdstl[pl.ds(i * 16, 16)] = tv + fl
        return 0
    lax.fori_loop(0, _CAP2 // 16, _meta, 0)

    t16[...] = jnp.full((16,), 1, jnp.int32) * total
    pltpu.sync_copy(seidl, seid_hbm.at[b])
    pltpu.sync_copy(sdstl.at[pl.ds(0, _CAP2)], meta_hbm.at[b])
    pltpu.sync_copy(t16, tot16_hbm.at[b])
    pltpu.sync_copy(cntf, cntw_hbm.at[b])


def _sortlocal(peid, pdst, offs, cnt16):
    return pl.kernel(
        _sortlocal_body,
        out_type=(
            jax.ShapeDtypeStruct((_NW, _CAP2), jnp.int32),
            jax.ShapeDtypeStruct((_NW, _CAP2), jnp.int32),
            jax.ShapeDtypeStruct((_NW, 16), jnp.int32),
            jax.ShapeDtypeStruct((_NW, 320), jnp.float32),
        ),
        mesh=_mesh(),
        compiler_params=_SC_PARAMS,
        scratch_types=[
            pltpu.VMEM((512,), jnp.int32),
            pltpu.VMEM((512,), jnp.int32),
            pltpu.VMEM((_NPW * 16,), jnp.int32),
            pltpu.VMEM((_NPW * 16,), jnp.int32),
            pltpu.VMEM((320,), jnp.float32),
            pltpu.VMEM((_CAP2,), jnp.int32),
            pltpu.VMEM((_CAP2 + 16,), jnp.int32),
            pltpu.VMEM((32,), jnp.int32),
            pltpu.VMEM((16,), jnp.int32),
            pltpu.VMEM((16,), jnp.int32),
        ],
    )(peid, pdst, offs, cnt16)


def _edge_gather_body(pd_hbm, ps_hbm, dst_hbm, src_hbm, g_hbm,
                      dstb, srcb, bufa, bufb, sema, semb, semw):
    w = _wid()
    ebase = w * _EPW2
    nblk = _EPW2 // _BLK

    pltpu.sync_copy(dst_hbm.at[pl.ds(ebase, _EPW2)], dstb)
    pltpu.sync_copy(src_hbm.at[pl.ds(ebase, _EPW2)], srcb)

    def _issue(t, q):
        for k in range(_BLK // _CH):
            off = pl.multiple_of(t * _BLK + k * _CH, _CH)
            sl = pl.ds(k * _CH, _CH)
            pltpu.async_copy(pd_hbm.at[dstb.at[pl.ds(off, _CH)]],
                             bufa.at[q].at[sl], sema.at[q])
            pltpu.async_copy(ps_hbm.at[srcb.at[pl.ds(off, _CH)]],
                             bufb.at[q].at[sl], semb.at[q])

    def _drain(q):
        for k in range(_BLK // _CH):
            sl = pl.ds(k * _CH, _CH)
            pltpu.make_async_copy(pd_hbm.at[dstb.at[pl.ds(0, _CH)]],
                                  bufa.at[q].at[sl], sema.at[q]).wait()
            pltpu.make_async_copy(ps_hbm.at[srcb.at[pl.ds(0, _CH)]],
                                  bufb.at[q].at[sl], semb.at[q]).wait()

    _issue(0, 0)
    _issue(1, 1)

    def _step(t, _):
        q = lax.rem(t, 3)

        @pl.when(t + 2 < nblk)
        def _():
            nq = lax.rem(t + 2, 3)
            @pl.when(t >= 1)
            def _():
                pltpu.make_async_copy(bufa.at[nq], g_hbm.at[pl.ds(ebase, _BLK)],
                                      semw.at[nq]).wait()
            _issue(t + 2, nq)

        _drain(q)

        def _row(r, _):
            for j in range(4):
                sl = pl.ds(j * 16, 16)
                bufa[q, r, sl] = bufa[q, r, sl] + bufb[q, r, sl]
            return 0
        lax.fori_loop(0, _BLK, _row, 0)
        pltpu.async_copy(bufa.at[q], g_hbm.at[pl.ds(ebase + t * _BLK, _BLK)],
                         semw.at[q])
        return 0
    lax.fori_loop(0, nblk, _step, 0)
    pltpu.make_async_copy(bufa.at[0], g_hbm.at[pl.ds(ebase, _BLK)], semw.at[0]).wait()
    pltpu.make_async_copy(bufa.at[1], g_hbm.at[pl.ds(ebase, _BLK)], semw.at[1]).wait()
    pltpu.make_async_copy(bufa.at[2], g_hbm.at[pl.ds(ebase, _BLK)], semw.at[2]).wait()


def _edge_gather(pd, ps, dst, src):
    return pl.kernel(
        _edge_gather_body,
        out_type=jax.ShapeDtypeStruct((_EP, _H), jnp.float32),
        mesh=_mesh(),
        compiler_params=_SC_PARAMS,
        scratch_types=[
            pltpu.VMEM((_EPW2,), jnp.int32),
            pltpu.VMEM((_EPW2,), jnp.int32),
            pltpu.VMEM((3, _BLK, _H), jnp.float32),
            pltpu.VMEM((3, _BLK, _H), jnp.float32),
            pltpu.SemaphoreType.DMA((3,)),
            pltpu.SemaphoreType.DMA((3,)),
            pltpu.SemaphoreType.DMA((3,)),
        ],
    )(pd, ps, dst, src)


def _aggregate_body(msg_hbm, seid_hbm, meta_hbm, tot16_hbm,
                    ssum_hbm, ssq_hbm, smx_hbm, smn_hbm,
                    accs, accq, accx, accn, seidl, metal, gbuf, t16, sem):
    b = _wid()
    nbase = b * _NPW
    zf = jnp.zeros((16,), jnp.float32)
    negv = jnp.full((16,), _NEG, jnp.float32)
    posv = jnp.full((16,), _POS, jnp.float32)

    def _init(i, _):
        for j in range(4):
            sl = pl.ds(j * 16, 16)
            accs[i, sl] = zf
            accq[i, sl] = zf
            accx[i, sl] = negv
            accn[i, sl] = posv
        return 0
    lax.fori_loop(0, _NPW, _init, 0)

    pltpu.sync_copy(seid_hbm.at[b], seidl)
    pltpu.sync_copy(meta_hbm.at[b], metal)
    pltpu.sync_copy(tot16_hbm.at[b], t16)
    total = t16[...][0]
    nblk = lax.div(total + _CH - 1, _CH)

    def _issue(t, q):
        off = pl.multiple_of(t * _CH, _CH)
        pltpu.async_copy(msg_hbm.at[seidl.at[pl.ds(off, _CH)]], gbuf.at[q], sem.at[q])

    def _drain(q):
        pltpu.make_async_copy(msg_hbm.at[seidl.at[pl.ds(0, _CH)]],
                              gbuf.at[q], sem.at[q]).wait()

    @pl.when(nblk > 0)
    def _():
        _issue(0, 0)

        @pl.when(nblk > 1)
        def _():
            _issue(1, 1)

        def _blk(t, _):
            q = lax.rem(t, 3)

            @pl.when(t + 2 < nblk)
            def _():
                _issue(t + 2, lax.rem(t + 2, 3))

            _drain(q)

            def _edge_any(e, regs, masked):
                (s0, s1, s2, s3, q0, q1, q2, q3,
                 x0, x1, x2, x3, n0, n1, n2, n3) = regs
                idx = t * _CH + e
                mv = plsc.load_gather(metal, [jnp.full((16,), 0, jnp.int32) + idx])[0]
                tnode = jnp.minimum(mv & 0xFFFF, _NPW - 1)
                fl = (mv >= (1 << 16)) | (e == _CH - 1)
                r0 = gbuf[q, e, pl.ds(0, 16)]
                r1 = gbuf[q, e, pl.ds(16, 16)]
                r2 = gbuf[q, e, pl.ds(32, 16)]
                r3 = gbuf[q, e, pl.ds(48, 16)]
                if masked:
                    ok = idx < total
                    z0 = jnp.where(ok, r0, 0.0)
                    z1 = jnp.where(ok, r1, 0.0)
                    z2 = jnp.where(ok, r2, 0.0)
                    z3 = jnp.where(ok, r3, 0.0)
                    m0 = jnp.where(ok, r0, _NEG)
                    m1 = jnp.where(ok, r1, _NEG)
                    m2 = jnp.where(ok, r2, _NEG)
                    m3 = jnp.where(ok, r3, _NEG)
                    p0 = jnp.where(ok, r0, _POS)
                    p1 = jnp.where(ok, r1, _POS)
                    p2 = jnp.where(ok, r2, _POS)
                    p3 = jnp.where(ok, r3, _POS)
                else:
                    z0, z1, z2, z3 = r0, r1, r2, r3
                    m0, m1, m2, m3 = r0, r1, r2, r3
                    p0, p1, p2, p3 = r0, r1, r2, r3
                s0 = s0 + z0
                s1 = s1 + z1
                s2 = s2 + z2
                s3 = s3 + z3
                q0 = q0 + z0 * z0
                q1 = q1 + z1 * z1
                q2 = q2 + z2 * z2
                q3 = q3 + z3 * z3
                x0 = jnp.maximum(x0, m0)
                x1 = jnp.maximum(x1, m1)
                x2 = jnp.maximum(x2, m2)
                x3 = jnp.maximum(x3, m3)
                n0 = jnp.minimum(n0, p0)
                n1 = jnp.minimum(n1, p1)
                n2 = jnp.minimum(n2, p2)
                n3 = jnp.minimum(n3, p3)

                @pl.when(fl)
                def _():
                    svs = (s0, s1, s2, s3)
                    qvs = (q0, q1, q2, q3)
                    xvs = (x0, x1, x2, x3)
                    nvs = (n0, n1, n2, n3)
                    for j in range(4):
                        sl = pl.ds(j * 16, 16)
                        accs[tnode, sl] = accs[tnode, sl] + svs[j]
                        accq[tnode, sl] = accq[tnode, sl] + qvs[j]
                        accx[tnode, sl] = jnp.maximum(accx[tnode, sl], xvs[j])
                        accn[tnode, sl] = jnp.minimum(accn[tnode, sl], nvs[j])

                s0 = jnp.where(fl, 0.0, s0)
                s1 = jnp.where(fl, 0.0, s1)
                s2 = jnp.where(fl, 0.0, s2)
                s3 = jnp.where(fl, 0.0, s3)
                q0 = jnp.where(fl, 0.0, q0)
                q1 = jnp.where(fl, 0.0, q1)
                q2 = jnp.where(fl, 0.0, q2)
                q3 = jnp.where(fl, 0.0, q3)
                x0 = jnp.where(fl, _NEG, x0)
                x1 = jnp.where(fl, _NEG, x1)
                x2 = jnp.where(fl, _NEG, x2)
                x3 = jnp.where(fl, _NEG, x3)
                n0 = jnp.where(fl, _POS, n0)
                n1 = jnp.where(fl, _POS, n1)
                n2 = jnp.where(fl, _POS, n2)
                n3 = jnp.where(fl, _POS, n3)
                return (s0, s1, s2, s3, q0, q1, q2, q3,
                        x0, x1, x2, x3, n0, n1, n2, n3)

            init = (zf, zf, zf, zf, zf, zf, zf, zf,
                    negv, negv, negv, negv, posv, posv, posv, posv)

            @pl.when(t + 1 < nblk)
            def _():
                lax.fori_loop(0, _CH, lambda e, r: _edge_any(e, r, False), init)

            @pl.when(t + 1 >= nblk)
            def _():
                lax.fori_loop(0, _CH, lambda e, r: _edge_any(e, r, True), init)
            return 0
        lax.fori_loop(0, nblk, _blk, 0)

    pltpu.sync_copy(accs, ssum_hbm.at[pl.ds(nbase, _NPW)])
    pltpu.sync_copy(accq, ssq_hbm.at[pl.ds(nbase, _NPW)])
    pltpu.sync_copy(accx, smx_hbm.at[pl.ds(nbase, _NPW)])
    pltpu.sync_copy(accn, smn_hbm.at[pl.ds(nbase, _NPW)])


def _aggregate(msg, seid, meta, tot16):
    return pl.kernel(
        _aggregate_body,
        out_type=(
            jax.ShapeDtypeStruct((_NP, _H), jnp.float32),
            jax.ShapeDtypeStruct((_NP, _H), jnp.float32),
            jax.ShapeDtypeStruct((_NP, _H), jnp.float32),
            jax.ShapeDtypeStruct((_NP, _H), jnp.float32),
        ),
        mesh=_mesh(),
        compiler_params=_SC_PARAMS,
        scratch_types=[
            pltpu.VMEM((_NPW, _H), jnp.float32),
            pltpu.VMEM((_NPW, _H), jnp.float32),
            pltpu.VMEM((_NPW, _H), jnp.float32),
            pltpu.VMEM((_NPW, _H), jnp.float32),
            pltpu.VMEM((_CAP2,), jnp.int32),
            pltpu.VMEM((_CAP2,), jnp.int32),
            pltpu.VMEM((3, _CH, _H), jnp.float32),
            pltpu.VMEM((16,), jnp.int32),
            pltpu.SemaphoreType.DMA((3,)),
        ],
    )(msg, seid, meta, tot16)


def _proj_body(h_ref, wd_ref, ws_ref, b1_ref, pd_ref, ps_ref):
    h = h_ref[...]
    pd_ref[...] = jnp.dot(h, wd_ref[...], preferred_element_type=jnp.float32) + b1_ref[...]
    ps_ref[...] = jnp.dot(h, ws_ref[...], preferred_element_type=jnp.float32)


def _proj(h, wd, ws, b1):
    return pl.pallas_call(
        _proj_body,
        out_shape=(jax.ShapeDtypeStruct((_NP, _H), jnp.float32),
                   jax.ShapeDtypeStruct((_NP, _H), jnp.float32)),
    )(h, wd, ws, b1.reshape(1, _H))


_BE2 = 768


def _edge_mlp0_body(g_ref, ea_ref, we_ref, w2_ref, b2_ref, out_ref):
    g2 = g_ref[...].reshape(_BE2, 128)
    pre = g2 + jnp.dot(ea_ref[...], we_ref[...], preferred_element_type=jnp.float32)
    h = jnp.maximum(pre, 0.0)
    m = jnp.dot(h, w2_ref[...], preferred_element_type=jnp.float32) + b2_ref[...]
    out_ref[...] = m.reshape(_BE2 * 128)


def _edge_mlp0(g1, ea2, we2, w22, b22):
    grid = _EP // (2 * _BE2)
    return pl.pallas_call(
        _edge_mlp0_body,
        out_shape=jax.ShapeDtypeStruct((_EP * _H,), jnp.float32),
        grid=(grid,),
        in_specs=[
            pl.BlockSpec((_BE2 * 128,), lambda i: (i,)),
            pl.BlockSpec((_BE2, 32), lambda i: (i, 0)),
            pl.BlockSpec((32, 128), lambda i: (0, 0)),
            pl.BlockSpec((128, 128), lambda i: (0, 0)),
            pl.BlockSpec((1, 128), lambda i: (0, 0)),
        ],
        out_specs=pl.BlockSpec((_BE2 * 128,), lambda i: (i,)),
    )(g1, ea2, we2, w22, b22.reshape(1, 128))


def _edge_mlp1_body(g_ref, ea_ref, we_ref, w2_ref, b2_ref, out_ref):
    g2 = g_ref[...].reshape(_BE2, 128)
    ea2 = ea_ref[...].reshape(_BE2, 128)
    pre = g2 + jnp.dot(ea2, we_ref[...], preferred_element_type=jnp.float32)
    h = jnp.maximum(pre, 0.0)
    m = jnp.dot(h, w2_ref[...], preferred_element_type=jnp.float32) + b2_ref[...]
    out_ref[...] = m.reshape(_BE2 * 128)


def _edge_mlp1(g1, ea1, we2, w22, b22):
    grid = _EP // (2 * _BE2)
    return pl.pallas_call(
        _edge_mlp1_body,
        out_shape=jax.ShapeDtypeStruct((_EP * _H,), jnp.float32),
        grid=(grid,),
        in_specs=[
            pl.BlockSpec((_BE2 * 128,), lambda i: (i,)),
            pl.BlockSpec((_BE2 * 128,), lambda i: (i,)),
            pl.BlockSpec((128, 128), lambda i: (0, 0)),
            pl.BlockSpec((128, 128), lambda i: (0, 0)),
            pl.BlockSpec((1, 128), lambda i: (0, 0)),
        ],
        out_specs=pl.BlockSpec((_BE2 * 128,), lambda i: (i,)),
    )(g1, ea1, we2, w22, b22.reshape(1, 128))


def _update_body(ssum_ref, ssq_ref, smx_ref, smn_ref, cnt_ref, h_ref,
                 um_ref, un_ref, ux_ref, us_ref, uh_ref, ub1_ref,
                 w2_ref, ub2_ref, out_ref):
    cnt = cnt_ref[...]
    cntc = jnp.maximum(cnt, 1.0)
    mean = ssum_ref[...] / cntc
    msq = ssq_ref[...] / cntc
    std = jnp.sqrt(jnp.maximum(msq - mean * mean, 0.0) + 1e-5)
    pos = cnt > 0.0
    mx = jnp.where(pos, smx_ref[...], 0.0)
    mn = jnp.where(pos, smn_ref[...], 0.0)
    z = (jnp.dot(mean, um_ref[...], preferred_element_type=jnp.float32)
         + jnp.dot(mn, un_ref[...], preferred_element_type=jnp.float32)
         + jnp.dot(mx, ux_ref[...], preferred_element_type=jnp.float32)
         + jnp.dot(std, us_ref[...], preferred_element_type=jnp.float32)
         + jnp.dot(h_ref[...], uh_ref[...], preferred_element_type=jnp.float32)
         + ub1_ref[...])
    z = jnp.maximum(z, 0.0)
    out_ref[...] = jnp.dot(z, w2_ref[...], preferred_element_type=jnp.float32) + ub2_ref[...]


def _update(ssum, ssq, smx, smn, cnt2d, h, um, un, ux, us, uh, ub1, w2, ub2):
    return pl.pallas_call(
        _update_body,
        out_shape=jax.ShapeDtypeStruct((_NP, _H), jnp.float32),
    )(ssum, ssq, smx, smn, cnt2d, h, um, un, ux, us, uh,
      ub1.reshape(1, _H), w2, ub2.reshape(1, _H))


def _readout_body(h_ref, vb_ref, gf_ref, f1h_ref, f1g_ref, b1_ref,
                  w2_ref, b2_ref, w3_ref, b3_ref, out_ref):
    onehot = (vb_ref[...] == lax.broadcasted_iota(jnp.int32, (1, 16), 1).astype(jnp.float32)).astype(jnp.float32)
    g = jnp.dot(onehot, gf_ref[...], preferred_element_type=jnp.float32)
    y = (jnp.dot(h_ref[...], f1h_ref[...], preferred_element_type=jnp.float32)
         + jnp.dot(g, f1g_ref[...], preferred_element_type=jnp.float32)
         + b1_ref[...])
    y = jnp.maximum(y, 0.0)
    y = jnp.maximum(jnp.dot(y, w2_ref[...], preferred_element_type=jnp.float32) + b2_ref[...], 0.0)
    out_ref[...] = jnp.dot(y, w3_ref[...], preferred_element_type=jnp.float32) + b3_ref[...]


def _readout(h, vb16, gf, f1h, f1g, b1, w2, b2, w3p, b3p):
    return pl.pallas_call(
        _readout_body,
        out_shape=jax.ShapeDtypeStruct((_NP, 128), jnp.float32),
    )(h, vb16, gf, f1h, f1g, b1.reshape(1, _H), w2, b2.reshape(1, _H),
      w3p, b3p.reshape(1, 128))


def kernel(x, edge_index, edge_attr, global_features, vertex_batch_map,
           edge_batch_map, params):
    src = edge_index[0].astype(jnp.int32)
    dst = edge_index[1].astype(jnp.int32)

    peid, pdst, offs, cnt16 = _bucketize(dst)
    seid, meta, tot16, cntw = _sortlocal(peid, pdst, offs, cnt16)

    padidx = (jnp.arange(_EP - _E, dtype=jnp.int32) * 37) % _N
    dstp = jnp.concatenate([dst, padidx])
    srcp = jnp.concatenate([src, padidx])

    h = jnp.pad(x, ((0, _NP - _N), (0, 0)))
    ea2 = jnp.pad(edge_attr, ((0, _EP - _E), (0, 0))).reshape(_EP // 2, 32)
    ea1 = None

    cnt_full = cntw[:, :_NPW].reshape(_NP)
    cnt2d = jnp.broadcast_to(cnt_full[:, None], (_NP, _H))

    eye2 = jnp.eye(2, dtype=jnp.float32)
    for l in range(5):
        p = params
        pre = f'b{l}_'
        mW1 = p[pre + 'mW1']
        fdim = 128 if l == 0 else _H
        wd = mW1[:fdim]
        ws = mW1[fdim:2 * fdim]
        we = mW1[2 * fdim:]
        we2 = jnp.kron(eye2, we)
        w22 = jnp.kron(eye2, p[pre + 'mW2'])
        b22 = jnp.concatenate([p[pre + 'mb2'], p[pre + 'mb2']])
        pd, ps = _proj(h, wd, ws, p[pre + 'mb1'])
        g = _edge_gather(pd, ps, dstp, srcp)
        g1 = g.reshape(_EP * _H)
        if l == 0:
            msg1 = _edge_mlp0(g1, ea2, we2, w22, b22)
        else:
            msg1 = _edge_mlp1(g1, ea1, we2, w22, b22)
        msg = msg1.reshape(_EP, _H)
        ssum, ssq, smx, smn = _aggregate(msg, seid, meta, tot16)
        uW1 = p[pre + 'uW1']
        um = uW1[0:_H]
        un = uW1[_H:2 * _H]
        ux = uW1[2 * _H:3 * _H]
        us = uW1[3 * _H:4 * _H]
        uh = uW1[4 * _H:]
        h = _update(ssum, ssq, smx, smn, cnt2d, h, um, un, ux, us, uh,
                    p[pre + 'ub1'], p[pre + 'uW2'], p[pre + 'ub2'])
        ea1 = msg1

    vertex_embeddings = h[:_N]

    vbp = jnp.pad(vertex_batch_map.astype(jnp.float32), (0, _NP - _N))
    vb16 = jnp.broadcast_to(vbp[:, None], (_NP, 16))
    w3p = jnp.pad(params['fc3W'], ((0, 0), (0, 127)))
    b3p = jnp.pad(params['fc3b'], (0, 127))
    q = _readout(h, vb16, global_features, params['fc1W'][:_H],
                 params['fc1W'][_H:], params['fc1b'], params['fc2W'],
                 params['fc2b'], w3p, b3p)
    q_values = q[:_N, :1]
    return (vertex_embeddings, q_values)


# speedup vs baseline: 5.1703x; 1.0684x over previous
import jax
import jax.numpy as jnp
from jax import lax
from jax.experimental import pallas as pl
from jax.experimental.pallas import tpu as pltpu
from jax.experimental.pallas import tpu_sc as plsc

_N = 10000
_E = 320000
_H = 64
_NW = 32
_NPW = 313
_NP = _NW * _NPW
_EPW = _E // _NW
_EPAD = 11024
_BLK = 256
_EPW2 = 42 * _BLK
_EP = _NW * _EPW2
_CAP2 = 11264
_CH = 128
_NEG = -3.4e38
_POS = 3.4e38


def _wid():
    return lax.axis_index("s") * 2 + lax.axis_index("c")


def _mesh():
    return plsc.VectorSubcoreMesh(core_axis_name="c", subcore_axis_name="s")


_SC_PARAMS = pltpu.CompilerParams(needs_layout_passes=False, use_tc_tiling_on_sc=False)


def _bucketize_body(dst_hbm, peid_hbm, pdst_hbm, offs_hbm, cnt16_hbm,
                    stage, hist, cur, offsv, leid, ldst):
    w = _wid()
    base = w * _EPW
    lanes = lax.iota(jnp.int32, 16)
    zeros16 = jnp.zeros((16,), jnp.int32)
    ones16 = jnp.ones((16,), jnp.int32)

    def _zero_hist(i, _):
        hist[pl.ds(i * 16, 16)] = zeros16
        return 0
    lax.fori_loop(0, 32, _zero_hist, 0)

    def _zero_lists(i, _):
        leid[pl.ds(i * 16, 16)] = zeros16
        ldst[pl.ds(i * 16, 16)] = zeros16
        return 0
    lax.fori_loop(0, _EPAD // 16, _zero_lists, 0)

    def _chunk1(c, _):
        pltpu.sync_copy(dst_hbm.at[pl.ds(base + c * 2000, 2000)], stage)
        def _vreg(k, _):
            v = stage[pl.ds(k * 16, 16)]
            bkt = lax.div(v, _NPW)
            plsc.addupdate_scatter(hist, [bkt * 16 + lanes], ones16)
            return 0
        lax.fori_loop(0, 125, _vreg, 0)
        return 0
    lax.fori_loop(0, 5, _chunk1, 0)

    carry = jnp.int32(0)
    for b in range(32):
        hv = hist[pl.ds(b * 16, 16)]
        cs = plsc.cumsum(hv)
        cur[pl.ds(b * 16, 16)] = cs - hv + carry
        plsc.store_scatter(offsv, [jnp.full((16,), b, jnp.int32)],
                           jnp.full((16,), 1, jnp.int32) * carry,
                           mask=lanes == 0)
        carry = lax.div(carry + cs[15] + 15, 16) * 16

    def _chunk2(c, _):
        pltpu.sync_copy(dst_hbm.at[pl.ds(base + c * 2000, 2000)], stage)
        def _vreg(k, _):
            v = stage[pl.ds(k * 16, 16)]
            bkt = lax.div(v, _NPW)
            key = bkt * 16 + lanes
            pos = plsc.load_gather(cur, [key])
            eid = jnp.full((16,), base + c * 2000 + k * 16, jnp.int32) + lanes
            plsc.store_scatter(leid, [pos], eid)
            plsc.store_scatter(ldst, [pos], v)
            plsc.store_scatter(cur, [key], pos + 1)
            return 0
        lax.fori_loop(0, 125, _vreg, 0)
        return 0
    lax.fori_loop(0, 5, _chunk2, 0)

    pltpu.sync_copy(leid, peid_hbm.at[w])
    pltpu.sync_copy(ldst, pdst_hbm.at[w])
    pltpu.sync_copy(offsv, offs_hbm.at[w])
    pltpu.sync_copy(hist, cnt16_hbm.at[w])


def _bucketize(dst):
    return pl.kernel(
        _bucketize_body,
        out_type=(
            jax.ShapeDtypeStruct((_NW, _EPAD), jnp.int32),
            jax.ShapeDtypeStruct((_NW, _EPAD), jnp.int32),
            jax.ShapeDtypeStruct((_NW, 32), jnp.int32),
            jax.ShapeDtypeStruct((_NW, 512), jnp.int32),
        ),
        mesh=_mesh(),
        compiler_params=_SC_PARAMS,
        scratch_types=[
            pltpu.VMEM((2000,), jnp.int32),
            pltpu.VMEM((512,), jnp.int32),
            pltpu.VMEM((512,), jnp.int32),
            pltpu.VMEM((32,), jnp.int32),
            pltpu.VMEM((_EPAD,), jnp.int32),
            pltpu.VMEM((_EPAD,), jnp.int32),
        ],
    )(dst)


def _sortlocal_body(peid_hbm, pdst_hbm, offs_hbm, cnt16_hbm,
                    seid_hbm, meta_hbm, tot16_hbm, cntw_hbm,
                    eidc, dstc, hist, cur, cntf, seidl, sdstl, offsv, c16, t16):
    b = _wid()
    nbase = b * _NPW
    lanes = lax.iota(jnp.int32, 16)
    zeros16 = jnp.zeros((16,), jnp.int32)
    ones16 = jnp.ones((16,), jnp.int32)

    def _zh(i, _):
        hist[pl.ds(i * 16, 16)] = zeros16
        return 0
    lax.fori_loop(0, _NPW, _zh, 0)

    def _zs(i, _):
        seidl[pl.ds(i * 16, 16)] = zeros16
        sdstl[pl.ds(i * 16, 16)] = zeros16
        return 0
    lax.fori_loop(0, (_CAP2 + 16) // 16, _zs, 0)

    for w in range(_NW):
        pltpu.sync_copy(offs_hbm.at[w], offsv)
        start = pl.multiple_of(
            plsc.load_gather(offsv, [jnp.full((16,), b, jnp.int32)])[0], 16)
        pltpu.sync_copy(cnt16_hbm.at[w].at[pl.ds(pl.multiple_of(b * 16, 16), 16)], c16)
        cnt_wb = jnp.sum(c16[...])
        nch = lax.div(cnt_wb + 511, 512)

        def _chunk(c, _):
            off = pl.multiple_of(start + c * 512, 16)
            pltpu.sync_copy(pdst_hbm.at[w].at[pl.ds(off, 512)], dstc)
            def _vreg(k, _):
                dv = dstc[pl.ds(k * 16, 16)]
                tv = jnp.clip(dv - nbase, 0, _NPW - 1)
                inr = (c * 512 + k * 16 + lanes) < cnt_wb
                plsc.addupdate_scatter(hist, [tv * 16 + lanes], ones16, mask=inr)
                return 0
            lax.fori_loop(0, 32, _vreg, 0)
            return 0
        lax.fori_loop(0, nch, _chunk, 0)

    def _scan(i, carry):
        hv = hist[pl.ds(i * 16, 16)]
        cs = plsc.cumsum(hv)
        cur[pl.ds(i * 16, 16)] = cs - hv + carry
        plsc.store_scatter(cntf, [jnp.full((16,), 0, jnp.int32) + i],
                           jnp.zeros((16,), jnp.float32) + cs[15].astype(jnp.float32),
                           mask=lanes == 0)
        return carry + cs[15]
    total = lax.fori_loop(0, _NPW, _scan, jnp.int32(0))

    for w in range(_NW):
        pltpu.sync_copy(offs_hbm.at[w], offsv)
        start = pl.multiple_of(
            plsc.load_gather(offsv, [jnp.full((16,), b, jnp.int32)])[0], 16)
        pltpu.sync_copy(cnt16_hbm.at[w].at[pl.ds(pl.multiple_of(b * 16, 16), 16)], c16)
        cnt_wb = jnp.sum(c16[...])
        nch = lax.div(cnt_wb + 511, 512)

        def _chunk(c, _):
            off = pl.multiple_of(start + c * 512, 16)
            pltpu.sync_copy(peid_hbm.at[w].at[pl.ds(off, 512)], eidc)
            pltpu.sync_copy(pdst_hbm.at[w].at[pl.ds(off, 512)], dstc)
            def _vreg(k, _):
                dv = dstc[pl.ds(k * 16, 16)]
                ev = eidc[pl.ds(k * 16, 16)]
                tv = jnp.clip(dv - nbase, 0, _NPW - 1)
                key = tv * 16 + lanes
                inr = (c * 512 + k * 16 + lanes) < cnt_wb
                pos = plsc.load_gather(cur, [key])
                plsc.store_scatter(seidl, [pos], ev, mask=inr)
                plsc.store_scatter(sdstl, [pos], dv, mask=inr)
                plsc.store_scatter(cur, [key], pos + 1, mask=inr)
                return 0
            lax.fori_loop(0, 32, _vreg, 0)
            return 0
        lax.fori_loop(0, nch, _chunk, 0)

    def _meta(i, _):
        dv = sdstl[pl.ds(i * 16, 16)]
        dn = plsc.load_gather(sdstl, [i * 16 + 1 + lanes])
        tv = jnp.clip(dv - nbase, 0, _NPW - 1)
        fl = jnp.where(dv != dn, jnp.int32(1 << 16), jnp.int32(0))
        sdstl[pl.ds(i * 16, 16)] = tv + fl
        return 0
    lax.fori_loop(0, _CAP2 // 16, _meta, 0)

    t16[...] = jnp.full((16,), 1, jnp.int32) * total
    pltpu.sync_copy(seidl, seid_hbm.at[b])
    pltpu.sync_copy(sdstl.at[pl.ds(0, _CAP2)], meta_hbm.at[b])
    pltpu.sync_copy(t16, tot16_hbm.at[b])
    pltpu.sync_copy(cntf, cntw_hbm.at[b])


def _sortlocal(peid, pdst, offs, cnt16):
    return pl.kernel(
        _sortlocal_body,
        out_type=(
            jax.ShapeDtypeStruct((_NW, _CAP2), jnp.int32),
            jax.ShapeDtypeStruct((_NW, _CAP2), jnp.int32),
            jax.ShapeDtypeStruct((_NW, 16), jnp.int32),
            jax.ShapeDtypeStruct((_NW, 320), jnp.float32),
        ),
        mesh=_mesh(),
        compiler_params=_SC_PARAMS,
        scratch_types=[
            pltpu.VMEM((512,), jnp.int32),
            pltpu.VMEM((512,), jnp.int32),
            pltpu.VMEM((_NPW * 16,), jnp.int32),
            pltpu.VMEM((_NPW * 16,), jnp.int32),
            pltpu.VMEM((320,), jnp.float32),
            pltpu.VMEM((_CAP2,), jnp.int32),
            pltpu.VMEM((_CAP2 + 16,), jnp.int32),
            pltpu.VMEM((32,), jnp.int32),
            pltpu.VMEM((16,), jnp.int32),
            pltpu.VMEM((16,), jnp.int32),
        ],
    )(peid, pdst, offs, cnt16)


def _edge_gather_body(pd_hbm, ps_hbm, dst_hbm, src_hbm, g_hbm,
                      dstb, srcb, bufa, bufb, sema, semb, semw):
    w = _wid()
    ebase = w * _EPW2
    nblk = _EPW2 // _BLK

    pltpu.sync_copy(dst_hbm.at[pl.ds(ebase, _EPW2)], dstb)
    pltpu.sync_copy(src_hbm.at[pl.ds(ebase, _EPW2)], srcb)

    def _issue(t, q):
        for k in range(_BLK // _CH):
            off = pl.multiple_of(t * _BLK + k * _CH, _CH)
            sl = pl.ds(k * _CH, _CH)
            pltpu.async_copy(pd_hbm.at[dstb.at[pl.ds(off, _CH)]],
                             bufa.at[q].at[sl], sema.at[q])
            pltpu.async_copy(ps_hbm.at[srcb.at[pl.ds(off, _CH)]],
                             bufb.at[q].at[sl], semb.at[q])

    def _drain(q):
        for k in range(_BLK // _CH):
            sl = pl.ds(k * _CH, _CH)
            pltpu.make_async_copy(pd_hbm.at[dstb.at[pl.ds(0, _CH)]],
                                  bufa.at[q].at[sl], sema.at[q]).wait()
            pltpu.make_async_copy(ps_hbm.at[srcb.at[pl.ds(0, _CH)]],
                                  bufb.at[q].at[sl], semb.at[q]).wait()

    _issue(0, 0)
    _issue(1, 1)

    def _step(t, _):
        q = lax.rem(t, 3)

        @pl.when(t + 2 < nblk)
        def _():
            nq = lax.rem(t + 2, 3)
            @pl.when(t >= 1)
            def _():
                pltpu.make_async_copy(bufa.at[nq], g_hbm.at[pl.ds(ebase, _BLK)],
                                      semw.at[nq]).wait()
            _issue(t + 2, nq)

        _drain(q)

        def _row(r, _):
            for j in range(4):
                sl = pl.ds(j * 16, 16)
                bufa[q, r, sl] = bufa[q, r, sl] + bufb[q, r, sl]
            return 0
        lax.fori_loop(0, _BLK, _row, 0)
        pltpu.async_copy(bufa.at[q], g_hbm.at[pl.ds(ebase + t * _BLK, _BLK)],
                         semw.at[q])
        return 0
    lax.fori_loop(0, nblk, _step, 0)
    pltpu.make_async_copy(bufa.at[0], g_hbm.at[pl.ds(ebase, _BLK)], semw.at[0]).wait()
    pltpu.make_async_copy(bufa.at[1], g_hbm.at[pl.ds(ebase, _BLK)], semw.at[1]).wait()
    pltpu.make_async_copy(bufa.at[2], g_hbm.at[pl.ds(ebase, _BLK)], semw.at[2]).wait()


def _edge_gather(pd, ps, dst, src):
    return pl.kernel(
        _edge_gather_body,
        out_type=jax.ShapeDtypeStruct((_EP, _H), jnp.float32),
        mesh=_mesh(),
        compiler_params=_SC_PARAMS,
        scratch_types=[
            pltpu.VMEM((_EPW2,), jnp.int32),
            pltpu.VMEM((_EPW2,), jnp.int32),
            pltpu.VMEM((3, _BLK, _H), jnp.float32),
            pltpu.VMEM((3, _BLK, _H), jnp.float32),
            pltpu.SemaphoreType.DMA((3,)),
            pltpu.SemaphoreType.DMA((3,)),
            pltpu.SemaphoreType.DMA((3,)),
        ],
    )(pd, ps, dst, src)


def _aggregate_body(msg_hbm, seid_hbm, meta_hbm, tot16_hbm,
                    ssum_hbm, ssq_hbm, smx_hbm, smn_hbm,
                    accs, accq, accx, accn, seidl, metal, gbuf, t16, sem):
    b = _wid()
    nbase = b * _NPW
    zf = jnp.zeros((16,), jnp.float32)
    negv = jnp.full((16,), _NEG, jnp.float32)
    posv = jnp.full((16,), _POS, jnp.float32)

    def _init(i, _):
        for j in range(4):
            sl = pl.ds(j * 16, 16)
            accs[i, sl] = zf
            accq[i, sl] = zf
            accx[i, sl] = negv
            accn[i, sl] = posv
        return 0
    lax.fori_loop(0, _NPW, _init, 0)

    pltpu.sync_copy(seid_hbm.at[b], seidl)
    pltpu.sync_copy(meta_hbm.at[b], metal)
    pltpu.sync_copy(tot16_hbm.at[b], t16)
    total = t16[...][0]
    nblk = lax.div(total + _CH - 1, _CH)

    def _issue(t, q):
        off = pl.multiple_of(t * _CH, _CH)
        pltpu.async_copy(msg_hbm.at[seidl.at[pl.ds(off, _CH)]], gbuf.at[q], sem.at[q])

    def _drain(q):
        pltpu.make_async_copy(msg_hbm.at[seidl.at[pl.ds(0, _CH)]],
                              gbuf.at[q], sem.at[q]).wait()

    @pl.when(nblk > 0)
    def _():
        _issue(0, 0)

        @pl.when(nblk > 1)
        def _():
            _issue(1, 1)

        def _blk(t, _):
            q = lax.rem(t, 3)

            @pl.when(t + 2 < nblk)
            def _():
                _issue(t + 2, lax.rem(t + 2, 3))

            _drain(q)

            def _edge_any(e, regs, masked):
                (s0, s1, s2, s3, q0, q1, q2, q3,
                 x0, x1, x2, x3, n0, n1, n2, n3) = regs
                idx = t * _CH + e
                mv = plsc.load_gather(metal, [jnp.full((16,), 0, jnp.int32) + idx])[0]
                tnode = jnp.minimum(mv & 0xFFFF, _NPW - 1)
                fl = (mv >= (1 << 16)) | (e == _CH - 1)
                r0 = gbuf[q, e, pl.ds(0, 16)]
                r1 = gbuf[q, e, pl.ds(16, 16)]
                r2 = gbuf[q, e, pl.ds(32, 16)]
                r3 = gbuf[q, e, pl.ds(48, 16)]
                if masked:
                    ok = idx < total
                    z0 = jnp.where(ok, r0, 0.0)
                    z1 = jnp.where(ok, r1, 0.0)
                    z2 = jnp.where(ok, r2, 0.0)
                    z3 = jnp.where(ok, r3, 0.0)
                    m0 = jnp.where(ok, r0, _NEG)
                    m1 = jnp.where(ok, r1, _NEG)
                    m2 = jnp.where(ok, r2, _NEG)
                    m3 = jnp.where(ok, r3, _NEG)
                    p0 = jnp.where(ok, r0, _POS)
                    p1 = jnp.where(ok, r1, _POS)
                    p2 = jnp.where(ok, r2, _POS)
                    p3 = jnp.where(ok, r3, _POS)
                else:
                    z0, z1, z2, z3 = r0, r1, r2, r3
                    m0, m1, m2, m3 = r0, r1, r2, r3
                    p0, p1, p2, p3 = r0, r1, r2, r3
                s0 = s0 + z0
                s1 = s1 + z1
                s2 = s2 + z2
                s3 = s3 + z3
                q0 = q0 + z0 * z0
                q1 = q1 + z1 * z1
                q2 = q2 + z2 * z2
                q3 = q3 + z3 * z3
                x0 = jnp.maximum(x0, m0)
                x1 = jnp.maximum(x1, m1)
                x2 = jnp.maximum(x2, m2)
                x3 = jnp.maximum(x3, m3)
                n0 = jnp.minimum(n0, p0)
                n1 = jnp.minimum(n1, p1)
                n2 = jnp.minimum(n2, p2)
                n3 = jnp.minimum(n3, p3)

                @pl.when(fl)
                def _():
                    svs = (s0, s1, s2, s3)
                    qvs = (q0, q1, q2, q3)
                    xvs = (x0, x1, x2, x3)
                    nvs = (n0, n1, n2, n3)
                    for j in range(4):
                        sl = pl.ds(j * 16, 16)
                        accs[tnode, sl] = accs[tnode, sl] + svs[j]
                        accq[tnode, sl] = accq[tnode, sl] + qvs[j]
                        accx[tnode, sl] = jnp.maximum(accx[tnode, sl], xvs[j])
                        accn[tnode, sl] = jnp.minimum(accn[tnode, sl], nvs[j])

                s0 = jnp.where(fl, 0.0, s0)
                s1 = jnp.where(fl, 0.0, s1)
                s2 = jnp.where(fl, 0.0, s2)
                s3 = jnp.where(fl, 0.0, s3)
                q0 = jnp.where(fl, 0.0, q0)
                q1 = jnp.where(fl, 0.0, q1)
                q2 = jnp.where(fl, 0.0, q2)
                q3 = jnp.where(fl, 0.0, q3)
                x0 = jnp.where(fl, _NEG, x0)
                x1 = jnp.where(fl, _NEG, x1)
                x2 = jnp.where(fl, _NEG, x2)
                x3 = jnp.where(fl, _NEG, x3)
                n0 = jnp.where(fl, _POS, n0)
                n1 = jnp.where(fl, _POS, n1)
                n2 = jnp.where(fl, _POS, n2)
                n3 = jnp.where(fl, _POS, n3)
                return (s0, s1, s2, s3, q0, q1, q2, q3,
                        x0, x1, x2, x3, n0, n1, n2, n3)

            init = (zf, zf, zf, zf, zf, zf, zf, zf,
                    negv, negv, negv, negv, posv, posv, posv, posv)

            @pl.when(t + 1 < nblk)
            def _():
                lax.fori_loop(0, _CH, lambda e, r: _edge_any(e, r, False), init)

            @pl.when(t + 1 >= nblk)
            def _():
                lax.fori_loop(0, _CH, lambda e, r: _edge_any(e, r, True), init)
            return 0
        lax.fori_loop(0, nblk, _blk, 0)

    pltpu.sync_copy(accs, ssum_hbm.at[pl.ds(nbase, _NPW)])
    pltpu.sync_copy(accq, ssq_hbm.at[pl.ds(nbase, _NPW)])
    pltpu.sync_copy(accx, smx_hbm.at[pl.ds(nbase, _NPW)])
    pltpu.sync_copy(accn, smn_hbm.at[pl.ds(nbase, _NPW)])


def _aggregate(msg, seid, meta, tot16):
    return pl.kernel(
        _aggregate_body,
        out_type=(
            jax.ShapeDtypeStruct((_NP, _H), jnp.float32),
            jax.ShapeDtypeStruct((_NP, _H), jnp.float32),
            jax.ShapeDtypeStruct((_NP, _H), jnp.float32),
            jax.ShapeDtypeStruct((_NP, _H), jnp.float32),
        ),
        mesh=_mesh(),
        compiler_params=_SC_PARAMS,
        scratch_types=[
            pltpu.VMEM((_NPW, _H), jnp.float32),
            pltpu.VMEM((_NPW, _H), jnp.float32),
            pltpu.VMEM((_NPW, _H), jnp.float32),
            pltpu.VMEM((_NPW, _H), jnp.float32),
            pltpu.VMEM((_CAP2,), jnp.int32),
            pltpu.VMEM((_CAP2,), jnp.int32),
            pltpu.VMEM((3, _CH, _H), jnp.float32),
            pltpu.VMEM((16,), jnp.int32),
            pltpu.SemaphoreType.DMA((3,)),
        ],
    )(msg, seid, meta, tot16)


def _proj_body(h_ref, wd_ref, ws_ref, b1_ref, pd_ref, ps_ref):
    h = h_ref[...]
    pd_ref[...] = jnp.dot(h, wd_ref[...], preferred_element_type=jnp.float32) + b1_ref[...]
    ps_ref[...] = jnp.dot(h, ws_ref[...], preferred_element_type=jnp.float32)


def _proj(h, wd, ws, b1):
    return pl.pallas_call(
        _proj_body,
        out_shape=(jax.ShapeDtypeStruct((_NP, _H), jnp.float32),
                   jax.ShapeDtypeStruct((_NP, _H), jnp.float32)),
    )(h, wd, ws, b1.reshape(1, _H))


_BE2 = 1536


def _edge_mlp0_body(g_ref, ea_ref, we_ref, w2_ref, b2_ref, out_ref):
    g2 = g_ref[...].reshape(_BE2, 128)
    pre = g2 + jnp.dot(ea_ref[...], we_ref[...], preferred_element_type=jnp.float32)
    h = jnp.maximum(pre, 0.0)
    m = jnp.dot(h, w2_ref[...], preferred_element_type=jnp.float32) + b2_ref[...]
    out_ref[...] = m.reshape(_BE2 * 128)


def _edge_mlp0(g1, ea2, we2, w22, b22):
    grid = _EP // (2 * _BE2)
    return pl.pallas_call(
        _edge_mlp0_body,
        out_shape=jax.ShapeDtypeStruct((_EP * _H,), jnp.float32),
        grid=(grid,),
        in_specs=[
            pl.BlockSpec((_BE2 * 128,), lambda i: (i,)),
            pl.BlockSpec((_BE2, 32), lambda i: (i, 0)),
            pl.BlockSpec((32, 128), lambda i: (0, 0)),
            pl.BlockSpec((128, 128), lambda i: (0, 0)),
            pl.BlockSpec((1, 128), lambda i: (0, 0)),
        ],
        out_specs=pl.BlockSpec((_BE2 * 128,), lambda i: (i,)),
    )(g1, ea2, we2, w22, b22.reshape(1, 128))


def _edge_mlp1_body(g_ref, ea_ref, we_ref, w2_ref, b2_ref, out_ref):
    g2 = g_ref[...].reshape(_BE2, 128)
    ea2 = ea_ref[...].reshape(_BE2, 128)
    pre = g2 + jnp.dot(ea2, we_ref[...], preferred_element_type=jnp.float32)
    h = jnp.maximum(pre, 0.0)
    m = jnp.dot(h, w2_ref[...], preferred_element_type=jnp.float32) + b2_ref[...]
    out_ref[...] = m.reshape(_BE2 * 128)


def _edge_mlp1(g1, ea1, we2, w22, b22):
    grid = _EP // (2 * _BE2)
    return pl.pallas_call(
        _edge_mlp1_body,
        out_shape=jax.ShapeDtypeStruct((_EP * _H,), jnp.float32),
        grid=(grid,),
        in_specs=[
            pl.BlockSpec((_BE2 * 128,), lambda i: (i,)),
            pl.BlockSpec((_BE2 * 128,), lambda i: (i,)),
            pl.BlockSpec((128, 128), lambda i: (0, 0)),
            pl.BlockSpec((128, 128), lambda i: (0, 0)),
            pl.BlockSpec((1, 128), lambda i: (0, 0)),
        ],
        out_specs=pl.BlockSpec((_BE2 * 128,), lambda i: (i,)),
    )(g1, ea1, we2, w22, b22.reshape(1, 128))


def _update_body(ssum_ref, ssq_ref, smx_ref, smn_ref, cnt_ref, h_ref,
                 um_ref, un_ref, ux_ref, us_ref, uh_ref, ub1_ref,
                 w2_ref, ub2_ref, out_ref):
    cnt = cnt_ref[...]
    cntc = jnp.maximum(cnt, 1.0)
    mean = ssum_ref[...] / cntc
    msq = ssq_ref[...] / cntc
    std = jnp.sqrt(jnp.maximum(msq - mean * mean, 0.0) + 1e-5)
    pos = cnt > 0.0
    mx = jnp.where(pos, smx_ref[...], 0.0)
    mn = jnp.where(pos, smn_ref[...], 0.0)
    z = (jnp.dot(mean, um_ref[...], preferred_element_type=jnp.float32)
         + jnp.dot(mn, un_ref[...], preferred_element_type=jnp.float32)
         + jnp.dot(mx, ux_ref[...], preferred_element_type=jnp.float32)
         + jnp.dot(std, us_ref[...], preferred_element_type=jnp.float32)
         + jnp.dot(h_ref[...], uh_ref[...], preferred_element_type=jnp.float32)
         + ub1_ref[...])
    z = jnp.maximum(z, 0.0)
    out_ref[...] = jnp.dot(z, w2_ref[...], preferred_element_type=jnp.float32) + ub2_ref[...]


def _update(ssum, ssq, smx, smn, cnt2d, h, um, un, ux, us, uh, ub1, w2, ub2):
    return pl.pallas_call(
        _update_body,
        out_shape=jax.ShapeDtypeStruct((_NP, _H), jnp.float32),
    )(ssum, ssq, smx, smn, cnt2d, h, um, un, ux, us, uh,
      ub1.reshape(1, _H), w2, ub2.reshape(1, _H))


def _readout_body(h_ref, vb_ref, gf_ref, f1h_ref, f1g_ref, b1_ref,
                  w2_ref, b2_ref, w3_ref, b3_ref, out_ref):
    onehot = (vb_ref[...] == lax.broadcasted_iota(jnp.int32, (1, 16), 1).astype(jnp.float32)).astype(jnp.float32)
    g = jnp.dot(onehot, gf_ref[...], preferred_element_type=jnp.float32)
    y = (jnp.dot(h_ref[...], f1h_ref[...], preferred_element_type=jnp.float32)
         + jnp.dot(g, f1g_ref[...], preferred_element_type=jnp.float32)
         + b1_ref[...])
    y = jnp.maximum(y, 0.0)
    y = jnp.maximum(jnp.dot(y, w2_ref[...], preferred_element_type=jnp.float32) + b2_ref[...], 0.0)
    out_ref[...] = jnp.dot(y, w3_ref[...], preferred_element_type=jnp.float32) + b3_ref[...]


def _readout(h, vb16, gf, f1h, f1g, b1, w2, b2, w3p, b3p):
    return pl.pallas_call(
        _readout_body,
        out_shape=jax.ShapeDtypeStruct((_NP, 128), jnp.float32),
    )(h, vb16, gf, f1h, f1g, b1.reshape(1, _H), w2, b2.reshape(1, _H),
      w3p, b3p.reshape(1, 128))


def kernel(x, edge_index, edge_attr, global_features, vertex_batch_map,
           edge_batch_map, params):
    src = edge_index[0].astype(jnp.int32)
    dst = edge_index[1].astype(jnp.int32)

    peid, pdst, offs, cnt16 = _bucketize(dst)
    seid, meta, tot16, cntw = _sortlocal(peid, pdst, offs, cnt16)

    padidx = (jnp.arange(_EP - _E, dtype=jnp.int32) * 37) % _N
    dstp = jnp.concatenate([dst, padidx])
    srcp = jnp.concatenate([src, padidx])

    h = jnp.pad(x, ((0, _NP - _N), (0, 0)))
    ea2 = jnp.pad(edge_attr, ((0, _EP - _E), (0, 0))).reshape(_EP // 2, 32)
    ea1 = None

    cnt_full = cntw[:, :_NPW].reshape(_NP)
    cnt2d = jnp.broadcast_to(cnt_full[:, None], (_NP, _H))

    eye2 = jnp.eye(2, dtype=jnp.float32)
    for l in range(5):
        p = params
        pre = f'b{l}_'
        mW1 = p[pre + 'mW1']
        fdim = 128 if l == 0 else _H
        wd = mW1[:fdim]
        ws = mW1[fdim:2 * fdim]
        we = mW1[2 * fdim:]
        we2 = jnp.kron(eye2, we)
        w22 = jnp.kron(eye2, p[pre + 'mW2'])
        b22 = jnp.concatenate([p[pre + 'mb2'], p[pre + 'mb2']])
        pd, ps = _proj(h, wd, ws, p[pre + 'mb1'])
        g = _edge_gather(pd, ps, dstp, srcp)
        g1 = g.reshape(_EP * _H)
        if l == 0:
            msg1 = _edge_mlp0(g1, ea2, we2, w22, b22)
        else:
            msg1 = _edge_mlp1(g1, ea1, we2, w22, b22)
        msg = msg1.reshape(_EP, _H)
        ssum, ssq, smx, smn = _aggregate(msg, seid, meta, tot16)
        uW1 = p[pre + 'uW1']
        um = uW1[0:_H]
        un = uW1[_H:2 * _H]
        ux = uW1[2 * _H:3 * _H]
        us = uW1[3 * _H:4 * _H]
        uh = uW1[4 * _H:]
        h = _update(ssum, ssq, smx, smn, cnt2d, h, um, un, ux, us, uh,
                    p[pre + 'ub1'], p[pre + 'uW2'], p[pre + 'ub2'])
        ea1 = msg1

    vertex_embeddings = h[:_N]

    vbp = jnp.pad(vertex_batch_map.astype(jnp.float32), (0, _NP - _N))
    vb16 = jnp.broadcast_to(vbp[:, None], (_NP, 16))
    w3p = jnp.pad(params['fc3W'], ((0, 0), (0, 127)))
    b3p = jnp.pad(params['fc3b'], (0, 127))
    q = _readout(h, vb16, global_features, params['fc1W'][:_H],
                 params['fc1W'][_H:], params['fc1b'], params['fc2W'],
                 params['fc2b'], w3p, b3p)
    q_values = q[:_N, :1]
    return (vertex_embeddings, q_values)
